# Initial kernel scaffold; baseline (speedup 1.0000x reference)
import jax
import jax.numpy as jnp
from jax.experimental import pallas as pl


def kernel(atom_features, pe, spherical, edge_index, line_h, line_edge_index, crystal_atom_idx, params):
    raise NotImplementedError("write your pallas kernel here")


# speedup vs baseline: 1.0294x; 1.0294x over previous
import functools
import jax
import jax.numpy as jnp
import numpy as np
from jax.experimental import pallas as pl
from jax.experimental.pallas import tpu as pltpu

_N_CRYSTALS = 128


def _lin(n, lo, step):
    return lo + step * jax.lax.broadcasted_iota(jnp.int32, (1, n), 1).astype(jnp.float32)


def _edge_feat_kernel(sph_ref, w_ref, b_ref, out_ref):
    sph = sph_ref[...]
    d0 = sph[:, 0:1]
    d1 = sph[:, 1:2]
    d2 = sph[:, 2:3]
    f0 = jnp.exp(-((d0 - _lin(41, 0.0, 0.2)) ** 2) / (0.2 ** 2))
    f1 = jnp.exp(-((d1 - _lin(17, 0.0, 0.2)) ** 2) / (0.2 ** 2))
    f2 = jnp.exp(-((d2 - _lin(17, -3.2, 0.4)) ** 2) / (0.4 ** 2))
    f3 = (d0 > 8.0).astype(jnp.float32)
    feat = jnp.concatenate([f0, f1, f2, f3], axis=1)
    out_ref[...] = jnp.dot(feat, w_ref[...], preferred_element_type=jnp.float32) + b_ref[...]


def _edge_features(spherical, w, b):
    ne = spherical.shape[0]
    blk = 2048
    return pl.pallas_call(
        _edge_feat_kernel,
        grid=(ne // blk,),
        in_specs=[
            pl.BlockSpec((blk, 3), lambda i: (i, 0)),
            pl.BlockSpec((76, 76), lambda i: (0, 0)),
            pl.BlockSpec((76,), lambda i: (0,)),
        ],
        out_specs=pl.BlockSpec((blk, 76), lambda i: (i, 0)),
        out_shape=jax.ShapeDtypeStruct((ne, 76), jnp.float32),
    )(spherical, w, b)


def _line_feat_kernel(lh_ref, w_ref, b_ref, out_ref):
    d = lh_ref[...][:, None]
    feat = jnp.exp(-((d - _lin(30, -1.4, 0.1)) ** 2) / (0.1 ** 2))
    out_ref[...] = jnp.dot(feat, w_ref[...], preferred_element_type=jnp.float32) + b_ref[...]


def _line_features(line_h, w, b):
    nl = line_h.shape[0]
    blk = 4096
    return pl.pallas_call(
        _line_feat_kernel,
        grid=(nl // blk,),
        in_specs=[
            pl.BlockSpec((blk,), lambda i: (i,)),
            pl.BlockSpec((30, 30), lambda i: (0, 0)),
            pl.BlockSpec((30,), lambda i: (0,)),
        ],
        out_specs=pl.BlockSpec((blk, 30), lambda i: (i, 0)),
        out_shape=jax.ShapeDtypeStruct((nl, 30), jnp.float32),
    )(line_h, w, b)


def _cgconv(x, edge_index, edge_attr, p):
    src, dst = edge_index[0], edge_index[1]
    z = jnp.concatenate([x[dst], x[src], edge_attr], axis=1)
    gate = jax.nn.sigmoid(z @ p['Wf'] + p['bf'])
    core = jax.nn.softplus(z @ p['Ws'] + p['bs'])
    agg = jax.ops.segment_sum(gate * core, dst, num_segments=x.shape[0])
    return x + agg


def _gt_layer(x, edge_index, edge_attr, p, n_heads=8, d_head=32):
    N = x.shape[0]
    src, dst = edge_index[0], edge_index[1]
    q = (x @ p['Wq']).reshape(N, n_heads, d_head)
    k = (x @ p['Wk']).reshape(N, n_heads, d_head)
    v = (x @ p['Wv']).reshape(N, n_heads, d_head)
    e = (edge_attr @ p['We']).reshape(-1, n_heads, d_head)
    logits = jnp.sum(q[dst] * (k[src] + e), axis=-1) / jnp.sqrt(float(d_head))
    m = jax.ops.segment_max(logits, dst, num_segments=N)
    m = jnp.where(jnp.isfinite(m), m, 0.0)
    ex = jnp.exp(logits - m[dst])
    denom = jax.ops.segment_sum(ex, dst, num_segments=N)
    alpha = ex / (denom[dst] + 1e-9)
    out = jax.ops.segment_sum(alpha[:, :, None] * (v[src] + e), dst, num_segments=N)
    return x + out.reshape(N, n_heads * d_head) @ p['Wo']


def kernel(atom_features, pe, spherical, edge_index, line_h, line_edge_index, crystal_atom_idx, params):
    N = pe.shape[0]
    nbr = _edge_features(spherical, params['W_edge'], params['b_edge'])
    atom = params['embeddings'][atom_features[:, 0]]
    atom = atom @ params['W_emb'] + params['b_emb']
    pe_h = pe @ params['W_pe'] + params['b_pe']
    line = _line_features(line_h, params['W_line'], params['b_line'])
    for cl, cn in zip(params['line_convs'], params['convs']):
        nbr = _cgconv(nbr, line_edge_index, line, cl)
        atom = _cgconv(atom, edge_index, nbr, cn)
    atom = atom + pe_h
    atom = jax.nn.softplus(_gt_layer(atom, edge_index, nbr, params['gt']))
    counts = jax.ops.segment_sum(jnp.ones((N,), jnp.float32), crystal_atom_idx, num_segments=_N_CRYSTALS)
    crys = jax.ops.segment_sum(atom, crystal_atom_idx, num_segments=_N_CRYSTALS) / jnp.clip(counts, 1.0)[:, None]
    crys = jax.nn.softplus(crys)
    crys = crys @ params['W_cf'] + params['b_cf']
    for fc in params['fcs']:
        crys = jax.nn.softplus(crys)
        crys = crys @ fc['W'] + fc['b']
    crys = jax.nn.softplus(crys)
    out = crys @ params['W_out'] + params['b_out']
    return out


# speedup vs baseline: 1.9328x; 1.8776x over previous
import functools
import jax
import jax.numpy as jnp
from jax.experimental import pallas as pl
from jax.experimental.pallas import tpu as pltpu

_N_CRYSTALS = 128


def _lin(n, lo, step):
    return lo + step * jax.lax.broadcasted_iota(jnp.int32, (1, n), 1).astype(jnp.float32)


def _edge_feat_kernel(sph_ref, w_ref, b_ref, out_ref):
    sph = sph_ref[...]
    d0 = sph[:, 0:1]
    d1 = sph[:, 1:2]
    d2 = sph[:, 2:3]
    f0 = jnp.exp(-((d0 - _lin(41, 0.0, 0.2)) ** 2) / (0.2 ** 2))
    f1 = jnp.exp(-((d1 - _lin(17, 0.0, 0.2)) ** 2) / (0.2 ** 2))
    f2 = jnp.exp(-((d2 - _lin(17, -3.2, 0.4)) ** 2) / (0.4 ** 2))
    f3 = (d0 > 8.0).astype(jnp.float32)
    feat = jnp.concatenate([f0, f1, f2, f3], axis=1)
    out_ref[...] = jnp.dot(feat, w_ref[...], preferred_element_type=jnp.float32) + b_ref[...]


def _edge_features(spherical, w, b):
    ne = spherical.shape[0]
    blk = 2048
    return pl.pallas_call(
        _edge_feat_kernel,
        grid=(ne // blk,),
        in_specs=[
            pl.BlockSpec((blk, 3), lambda i: (i, 0)),
            pl.BlockSpec((76, 76), lambda i: (0, 0)),
            pl.BlockSpec((76,), lambda i: (0,)),
        ],
        out_specs=pl.BlockSpec((blk, 76), lambda i: (i, 0)),
        out_shape=jax.ShapeDtypeStruct((ne, 76), jnp.float32),
    )(spherical, w, b)


def _line_feat_kernel(lh_ref, w_ref, b_ref, out_ref):
    d = lh_ref[...][:, None]
    feat = jnp.exp(-((d - _lin(30, -1.4, 0.1)) ** 2) / (0.1 ** 2))
    out_ref[...] = jnp.dot(feat, w_ref[...], preferred_element_type=jnp.float32) + b_ref[...]


def _line_features(line_h, w, b):
    nl = line_h.shape[0]
    blk = 4096
    return pl.pallas_call(
        _line_feat_kernel,
        grid=(nl // blk,),
        in_specs=[
            pl.BlockSpec((blk,), lambda i: (i,)),
            pl.BlockSpec((30, 30), lambda i: (0, 0)),
            pl.BlockSpec((30,), lambda i: (0,)),
        ],
        out_specs=pl.BlockSpec((blk, 30), lambda i: (i, 0)),
        out_shape=jax.ShapeDtypeStruct((nl, 30), jnp.float32),
    )(line_h, w, b)


def _embed_kernel(af_ref, emb_ref, wemb_ref, bemb_ref, pe_ref, wpe_ref, bpe_ref,
                  atom_ref, peh_ref):
    af = af_ref[...]
    oh = (af == jax.lax.broadcasted_iota(jnp.int32, (1, 100), 1)).astype(jnp.float32)
    embw = jnp.dot(emb_ref[...], wemb_ref[...], preferred_element_type=jnp.float32)
    atom_ref[...] = jnp.dot(oh, embw, preferred_element_type=jnp.float32) + bemb_ref[...]
    peh_ref[...] = jnp.dot(pe_ref[...], wpe_ref[...], preferred_element_type=jnp.float32) + bpe_ref[...]


def _embed(atom_features, emb, wemb, bemb, pe, wpe, bpe):
    n = pe.shape[0]
    blk = 2000
    return pl.pallas_call(
        _embed_kernel,
        grid=(n // blk,),
        in_specs=[
            pl.BlockSpec((blk, 1), lambda i: (i, 0)),
            pl.BlockSpec((100, 92), lambda i: (0, 0)),
            pl.BlockSpec((92, 256), lambda i: (0, 0)),
            pl.BlockSpec((256,), lambda i: (0,)),
            pl.BlockSpec((blk, 40), lambda i: (i, 0)),
            pl.BlockSpec((40, 256), lambda i: (0, 0)),
            pl.BlockSpec((256,), lambda i: (0,)),
        ],
        out_specs=[
            pl.BlockSpec((blk, 256), lambda i: (i, 0)),
            pl.BlockSpec((blk, 256), lambda i: (i, 0)),
        ],
        out_shape=[
            jax.ShapeDtypeStruct((n, 256), jnp.float32),
            jax.ShapeDtypeStruct((n, 256), jnp.float32),
        ],
    )(atom_features.astype(jnp.int32), emb, wemb, bemb, pe, wpe, bpe)


def _conv_msg_kernel(xd_ref, xs_ref, e_ref, wf_ref, bf_ref, ws_ref, bs_ref, m_ref):
    z = jnp.concatenate([xd_ref[...], xs_ref[...], e_ref[...]], axis=1)
    gate = jax.nn.sigmoid(jnp.dot(z, wf_ref[...], preferred_element_type=jnp.float32) + bf_ref[...])
    core = jax.nn.softplus(jnp.dot(z, ws_ref[...], preferred_element_type=jnp.float32) + bs_ref[...])
    m_ref[...] = gate * core


def _conv_msgs(xd, xs, e, wf, bf, ws, bs, blk):
    ne, fx = xd.shape
    fe = e.shape[1]
    fz = 2 * fx + fe
    fo = wf.shape[1]
    return pl.pallas_call(
        _conv_msg_kernel,
        grid=(ne // blk,),
        in_specs=[
            pl.BlockSpec((blk, fx), lambda i: (i, 0)),
            pl.BlockSpec((blk, fx), lambda i: (i, 0)),
            pl.BlockSpec((blk, fe), lambda i: (i, 0)),
            pl.BlockSpec((fz, fo), lambda i: (0, 0)),
            pl.BlockSpec((fo,), lambda i: (0,)),
            pl.BlockSpec((fz, fo), lambda i: (0, 0)),
            pl.BlockSpec((fo,), lambda i: (0,)),
        ],
        out_specs=pl.BlockSpec((blk, fo), lambda i: (i, 0)),
        out_shape=jax.ShapeDtypeStruct((ne, fo), jnp.float32),
    )(xd, xs, e, wf, bf, ws, bs)


def _gt1_kernel(xd_ref, xs_ref, nbr_ref, wq_ref, wk_ref, wv_ref, we_ref, ex_ref, ve_ref):
    q = jnp.dot(xd_ref[...], wq_ref[...], preferred_element_type=jnp.float32)
    xs = xs_ref[...]
    k = jnp.dot(xs, wk_ref[...], preferred_element_type=jnp.float32)
    v = jnp.dot(xs, wv_ref[...], preferred_element_type=jnp.float32)
    e = jnp.dot(nbr_ref[...], we_ref[...], preferred_element_type=jnp.float32)
    s = q * (k + e)
    lane = jax.lax.broadcasted_iota(jnp.int32, (256, 8), 0)
    head = jax.lax.broadcasted_iota(jnp.int32, (256, 8), 1)
    msk = ((lane // 32) == head).astype(jnp.float32)
    logits = jnp.dot(s, msk, preferred_element_type=jnp.float32) * (1.0 / jnp.sqrt(32.0))
    ex_ref[...] = jnp.exp(logits)
    ve_ref[...] = v + e


def _gt1(xd, xs, nbr, wq, wk, wv, we, blk=2048):
    ne = xd.shape[0]
    return pl.pallas_call(
        _gt1_kernel,
        grid=(ne // blk,),
        in_specs=[
            pl.BlockSpec((blk, 256), lambda i: (i, 0)),
            pl.BlockSpec((blk, 256), lambda i: (i, 0)),
            pl.BlockSpec((blk, 76), lambda i: (i, 0)),
            pl.BlockSpec((256, 256), lambda i: (0, 0)),
            pl.BlockSpec((256, 256), lambda i: (0, 0)),
            pl.BlockSpec((256, 256), lambda i: (0, 0)),
            pl.BlockSpec((76, 256), lambda i: (0, 0)),
        ],
        out_specs=[
            pl.BlockSpec((blk, 8), lambda i: (i, 0)),
            pl.BlockSpec((blk, 256), lambda i: (i, 0)),
        ],
        out_shape=[
            jax.ShapeDtypeStruct((ne, 8), jnp.float32),
            jax.ShapeDtypeStruct((ne, 256), jnp.float32),
        ],
    )(xd, xs, nbr, wq, wk, wv, we)


def _gt2_kernel(ex_ref, dg_ref, ve_ref, m_ref):
    alpha = ex_ref[...] / (dg_ref[...] + 1e-9)
    lane = jax.lax.broadcasted_iota(jnp.int32, (8, 256), 1)
    head = jax.lax.broadcasted_iota(jnp.int32, (8, 256), 0)
    msk = ((lane // 32) == head).astype(jnp.float32)
    m_ref[...] = jnp.dot(alpha, msk, preferred_element_type=jnp.float32) * ve_ref[...]


def _gt2(ex, dg, ve, blk=2048):
    ne = ex.shape[0]
    return pl.pallas_call(
        _gt2_kernel,
        grid=(ne // blk,),
        in_specs=[
            pl.BlockSpec((blk, 8), lambda i: (i, 0)),
            pl.BlockSpec((blk, 8), lambda i: (i, 0)),
            pl.BlockSpec((blk, 256), lambda i: (i, 0)),
        ],
        out_specs=pl.BlockSpec((blk, 256), lambda i: (i, 0)),
        out_shape=jax.ShapeDtypeStruct((ne, 256), jnp.float32),
    )(ex, dg, ve)


def _add_kernel(a_ref, b_ref, o_ref):
    o_ref[...] = a_ref[...] + b_ref[...]


def _residual_add(a, b, blk=4096):
    n, f = a.shape
    g = (n + blk - 1) // blk
    return pl.pallas_call(
        _add_kernel,
        grid=(g,),
        in_specs=[pl.BlockSpec((blk, f), lambda i: (i, 0)),
                  pl.BlockSpec((blk, f), lambda i: (i, 0))],
        out_specs=pl.BlockSpec((blk, f), lambda i: (i, 0)),
        out_shape=jax.ShapeDtypeStruct((n, f), jnp.float32),
    )(a, b)


def _head_kernel(xg_ref, agg_ref, cidx_ref, wo_ref, wcf_ref, bcf_ref,
                 w1_ref, b1_ref, w2_ref, b2_ref, wout_ref, bout_ref,
                 out_ref, acc_ref, cnt_ref):
    i = pl.program_id(0)
    nblk = pl.num_programs(0)

    @pl.when(i == 0)
    def _init():
        acc_ref[...] = jnp.zeros_like(acc_ref)
        cnt_ref[...] = jnp.zeros_like(cnt_ref)

    x = jax.nn.softplus(
        xg_ref[...] + jnp.dot(agg_ref[...], wo_ref[...], preferred_element_type=jnp.float32))
    cid = cidx_ref[...]
    oh = (cid == jax.lax.broadcasted_iota(jnp.int32, (1, _N_CRYSTALS), 1)).astype(jnp.float32)
    acc_ref[...] += jax.lax.dot_general(oh, x, (((0,), (0,)), ((), ())),
                                        preferred_element_type=jnp.float32)
    ones = jnp.ones((x.shape[0], 8), jnp.float32)
    cnt_ref[...] += jax.lax.dot_general(oh, ones, (((0,), (0,)), ((), ())),
                                        preferred_element_type=jnp.float32)

    @pl.when(i == nblk - 1)
    def _finish():
        cnt = jnp.clip(cnt_ref[:, 0:1], 1.0, None)
        crys = jax.nn.softplus(acc_ref[...] / cnt)
        crys = jnp.dot(crys, wcf_ref[...], preferred_element_type=jnp.float32) + bcf_ref[...]
        crys = jax.nn.softplus(crys)
        crys = jnp.dot(crys, w1_ref[...], preferred_element_type=jnp.float32) + b1_ref[...]
        crys = jax.nn.softplus(crys)
        crys = jnp.dot(crys, w2_ref[...], preferred_element_type=jnp.float32) + b2_ref[...]
        crys = jax.nn.softplus(crys)
        out_ref[...] = jnp.dot(crys, wout_ref[...], preferred_element_type=jnp.float32) + bout_ref[...]


def _head(xg, agg, cidx, wo, wcf, bcf, fcs, wout, bout, blk=2000):
    n = xg.shape[0]
    return pl.pallas_call(
        _head_kernel,
        grid=(n // blk,),
        in_specs=[
            pl.BlockSpec((blk, 256), lambda i: (i, 0)),
            pl.BlockSpec((blk, 256), lambda i: (i, 0)),
            pl.BlockSpec((blk, 1), lambda i: (i, 0)),
            pl.BlockSpec((256, 256), lambda i: (0, 0)),
            pl.BlockSpec((256, 256), lambda i: (0, 0)),
            pl.BlockSpec((256,), lambda i: (0,)),
            pl.BlockSpec((256, 256), lambda i: (0, 0)),
            pl.BlockSpec((256,), lambda i: (0,)),
            pl.BlockSpec((256, 256), lambda i: (0, 0)),
            pl.BlockSpec((256,), lambda i: (0,)),
            pl.BlockSpec((256, 1), lambda i: (0, 0)),
            pl.BlockSpec((1,), lambda i: (0,)),
        ],
        out_specs=pl.BlockSpec((_N_CRYSTALS, 1), lambda i: (0, 0)),
        out_shape=jax.ShapeDtypeStruct((_N_CRYSTALS, 1), jnp.float32),
        scratch_shapes=[
            pltpu.VMEM((_N_CRYSTALS, 256), jnp.float32),
            pltpu.VMEM((_N_CRYSTALS, 8), jnp.float32),
        ],
    )(xg, agg, cidx.reshape(n, 1).astype(jnp.int32), wo, wcf, bcf,
      fcs[0]['W'], fcs[0]['b'], fcs[1]['W'], fcs[1]['b'], wout, bout)


def _gather(table, idx):
    return jnp.take(table, idx, axis=0)


def _scatter_add(msgs, idx, n_rows):
    return jax.ops.segment_sum(msgs, idx, num_segments=n_rows)


def kernel(atom_features, pe, spherical, edge_index, line_h, line_edge_index, crystal_atom_idx, params):
    n = pe.shape[0]
    ne = spherical.shape[0]
    src, dst = edge_index[0], edge_index[1]
    lsrc, ldst = line_edge_index[0], line_edge_index[1]

    nbr = _edge_features(spherical, params['W_edge'], params['b_edge'])
    atom, pe_h = _embed(atom_features, params['embeddings'], params['W_emb'], params['b_emb'],
                        pe, params['W_pe'], params['b_pe'])
    line = _line_features(line_h, params['W_line'], params['b_line'])

    for cl, cn in zip(params['line_convs'], params['convs']):
        nd = _gather(nbr, ldst)
        ns = _gather(nbr, lsrc)
        lmsg = _conv_msgs(nd, ns, line, cl['Wf'], cl['bf'], cl['Ws'], cl['bs'], blk=2048)
        nbr = _residual_add(nbr, _scatter_add(lmsg, ldst, ne))

        xd = _gather(atom, dst)
        xs = _gather(atom, src)
        msg = _conv_msgs(xd, xs, nbr, cn['Wf'], cn['bf'], cn['Ws'], cn['bs'], blk=2048)
        atom = _residual_add(atom, _scatter_add(msg, dst, n))

    xg = _residual_add(atom, pe_h)
    gxd = _gather(xg, dst)
    gxs = _gather(xg, src)
    ex, ve = _gt1(gxd, gxs, nbr, params['gt']['Wq'], params['gt']['Wk'],
                  params['gt']['Wv'], params['gt']['We'])
    denom = _scatter_add(ex, dst, n)
    dg = _gather(denom, dst)
    msg = _gt2(ex, dg, ve)
    agg = _scatter_add(msg, dst, n)

    return _head(xg, agg, crystal_atom_idx, params['gt']['Wo'], params['W_cf'],
                 params['b_cf'], params['fcs'], params['W_out'], params['b_out'])


# speedup vs baseline: 3.1803x; 1.6455x over previous
import functools
import jax
import jax.numpy as jnp
from jax import lax
from jax.experimental import pallas as pl
from jax.experimental.pallas import tpu as pltpu
from jax.experimental.pallas import tpu_sc as plsc

_N_CRYSTALS = 128
_SC_NC = 2
_SC_NS = 16
_SC_NW = _SC_NC * _SC_NS


def _lin(n, lo, step):
    return lo + step * jax.lax.broadcasted_iota(jnp.int32, (1, n), 1).astype(jnp.float32)


def _edge_feat_kernel(sph_ref, w_ref, b_ref, out_ref):
    sph = sph_ref[...]
    d0 = sph[:, 0:1]
    d1 = sph[:, 1:2]
    d2 = sph[:, 2:3]
    f0 = jnp.exp(-((d0 - _lin(41, 0.0, 0.2)) ** 2) / (0.2 ** 2))
    f1 = jnp.exp(-((d1 - _lin(17, 0.0, 0.2)) ** 2) / (0.2 ** 2))
    f2 = jnp.exp(-((d2 - _lin(17, -3.2, 0.4)) ** 2) / (0.4 ** 2))
    f3 = (d0 > 8.0).astype(jnp.float32)
    feat = jnp.concatenate([f0, f1, f2, f3], axis=1)
    r = jnp.dot(feat, w_ref[...], preferred_element_type=jnp.float32) + b_ref[...]
    out_ref[...] = jnp.concatenate(
        [r, jnp.zeros((r.shape[0], 52), jnp.float32)], axis=1)


def _edge_features(spherical, w, b):
    ne = spherical.shape[0]
    blk = 2048
    return pl.pallas_call(
        _edge_feat_kernel,
        grid=(pl.cdiv(ne, blk),),
        in_specs=[
            pl.BlockSpec((blk, 3), lambda i: (i, 0)),
            pl.BlockSpec((76, 76), lambda i: (0, 0)),
            pl.BlockSpec((76,), lambda i: (0,)),
        ],
        out_specs=pl.BlockSpec((blk, 128), lambda i: (i, 0)),
        out_shape=jax.ShapeDtypeStruct((ne, 128), jnp.float32),
    )(spherical, w, b)


def _line_feat_kernel(lh_ref, w_ref, b_ref, out_ref):
    d = lh_ref[...][:, None]
    feat = jnp.exp(-((d - _lin(30, -1.4, 0.1)) ** 2) / (0.1 ** 2))
    out_ref[...] = jnp.dot(feat, w_ref[...], preferred_element_type=jnp.float32) + b_ref[...]


def _line_features(line_h, w, b):
    nl = line_h.shape[0]
    blk = 4096
    return pl.pallas_call(
        _line_feat_kernel,
        grid=(nl // blk,),
        in_specs=[
            pl.BlockSpec((blk,), lambda i: (i,)),
            pl.BlockSpec((30, 30), lambda i: (0, 0)),
            pl.BlockSpec((30,), lambda i: (0,)),
        ],
        out_specs=pl.BlockSpec((blk, 30), lambda i: (i, 0)),
        out_shape=jax.ShapeDtypeStruct((nl, 30), jnp.float32),
    )(line_h, w, b)


def _embed_kernel(af_ref, emb_ref, wemb_ref, bemb_ref, pe_ref, wpe_ref, bpe_ref,
                  atom_ref, peh_ref):
    af = af_ref[...]
    oh = (af == jax.lax.broadcasted_iota(jnp.int32, (1, 100), 1)).astype(jnp.float32)
    embw = jnp.dot(emb_ref[...], wemb_ref[...], preferred_element_type=jnp.float32)
    atom_ref[...] = jnp.dot(oh, embw, preferred_element_type=jnp.float32) + bemb_ref[...]
    peh_ref[...] = jnp.dot(pe_ref[...], wpe_ref[...], preferred_element_type=jnp.float32) + bpe_ref[...]


def _embed(atom_features, emb, wemb, bemb, pe, wpe, bpe):
    n = pe.shape[0]
    blk = 2000
    return pl.pallas_call(
        _embed_kernel,
        grid=(n // blk,),
        in_specs=[
            pl.BlockSpec((blk, 1), lambda i: (i, 0)),
            pl.BlockSpec((100, 92), lambda i: (0, 0)),
            pl.BlockSpec((92, 256), lambda i: (0, 0)),
            pl.BlockSpec((256,), lambda i: (0,)),
            pl.BlockSpec((blk, 40), lambda i: (i, 0)),
            pl.BlockSpec((40, 256), lambda i: (0, 0)),
            pl.BlockSpec((256,), lambda i: (0,)),
        ],
        out_specs=[
            pl.BlockSpec((blk, 256), lambda i: (i, 0)),
            pl.BlockSpec((blk, 256), lambda i: (i, 0)),
        ],
        out_shape=[
            jax.ShapeDtypeStruct((n, 256), jnp.float32),
            jax.ShapeDtypeStruct((n, 256), jnp.float32),
        ],
    )(atom_features.astype(jnp.int32), emb, wemb, bemb, pe, wpe, bpe)


def _make_conv_msg_kernel(fx, fe, fo_pad):
    def _conv_msg_kernel(xd_ref, xs_ref, e_ref, wf_ref, bf_ref, ws_ref, bs_ref, m_ref):
        z = jnp.concatenate([xd_ref[:, :fx], xs_ref[:, :fx], e_ref[:, :fe]], axis=1)
        gate = jax.nn.sigmoid(jnp.dot(z, wf_ref[...], preferred_element_type=jnp.float32) + bf_ref[...])
        core = jax.nn.softplus(jnp.dot(z, ws_ref[...], preferred_element_type=jnp.float32) + bs_ref[...])
        m = gate * core
        if fo_pad > m.shape[1]:
            m = jnp.concatenate(
                [m, jnp.zeros((m.shape[0], fo_pad - m.shape[1]), jnp.float32)], axis=1)
        m_ref[...] = m
    return _conv_msg_kernel


def _conv_msgs(xd, xs, e, wf, bf, ws, bs, blk, fx, fe, fo_pad=None):
    epad = xd.shape[0]
    fxs = xd.shape[1]
    fes = e.shape[1]
    fz, fo = wf.shape
    fo_pad = fo if fo_pad is None else fo_pad
    return pl.pallas_call(
        _make_conv_msg_kernel(fx, fe, fo_pad),
        grid=(epad // blk,),
        in_specs=[
            pl.BlockSpec((blk, fxs), lambda i: (i, 0)),
            pl.BlockSpec((blk, fxs), lambda i: (i, 0)),
            pl.BlockSpec((blk, fes), lambda i: (i, 0)),
            pl.BlockSpec((fz, fo), lambda i: (0, 0)),
            pl.BlockSpec((fo,), lambda i: (0,)),
            pl.BlockSpec((fz, fo), lambda i: (0, 0)),
            pl.BlockSpec((fo,), lambda i: (0,)),
        ],
        out_specs=pl.BlockSpec((blk, fo_pad), lambda i: (i, 0)),
        out_shape=jax.ShapeDtypeStruct((epad, fo_pad), jnp.float32),
    )(xd, xs, e, wf, bf, ws, bs)


def _gt1_kernel(xd_ref, xs_ref, nbr_ref, wq_ref, wk_ref, wv_ref, we_ref, ex_ref, ve_ref):
    q = jnp.dot(xd_ref[...], wq_ref[...], preferred_element_type=jnp.float32)
    xs = xs_ref[...]
    k = jnp.dot(xs, wk_ref[...], preferred_element_type=jnp.float32)
    v = jnp.dot(xs, wv_ref[...], preferred_element_type=jnp.float32)
    e = jnp.dot(nbr_ref[:, :76], we_ref[...], preferred_element_type=jnp.float32)
    s = q * (k + e)
    lane = jax.lax.broadcasted_iota(jnp.int32, (256, 8), 0)
    head = jax.lax.broadcasted_iota(jnp.int32, (256, 8), 1)
    msk = ((lane // 32) == head).astype(jnp.float32)
    logits = jnp.dot(s, msk, preferred_element_type=jnp.float32) * (1.0 / jnp.sqrt(32.0))
    ex = jnp.exp(logits)
    ex_ref[...] = jnp.concatenate(
        [ex, jnp.zeros((ex.shape[0], 248), jnp.float32)], axis=1)
    ve_ref[...] = v + e


def _gt1(xd, xs, nbr, wq, wk, wv, we, blk=2048):
    epad = xd.shape[0]
    return pl.pallas_call(
        _gt1_kernel,
        grid=(epad // blk,),
        in_specs=[
            pl.BlockSpec((blk, 256), lambda i: (i, 0)),
            pl.BlockSpec((blk, 256), lambda i: (i, 0)),
            pl.BlockSpec((blk, 128), lambda i: (i, 0)),
            pl.BlockSpec((256, 256), lambda i: (0, 0)),
            pl.BlockSpec((256, 256), lambda i: (0, 0)),
            pl.BlockSpec((256, 256), lambda i: (0, 0)),
            pl.BlockSpec((76, 256), lambda i: (0, 0)),
        ],
        out_specs=[
            pl.BlockSpec((blk, 256), lambda i: (i, 0)),
            pl.BlockSpec((blk, 256), lambda i: (i, 0)),
        ],
        out_shape=[
            jax.ShapeDtypeStruct((epad, 256), jnp.float32),
            jax.ShapeDtypeStruct((epad, 256), jnp.float32),
        ],
    )(xd, xs, nbr, wq, wk, wv, we)


def _gt2_kernel(ex_ref, dg_ref, ve_ref, m_ref):
    alpha = ex_ref[:, 0:8] / (dg_ref[:, 0:8] + 1e-9)
    lane = jax.lax.broadcasted_iota(jnp.int32, (8, 256), 1)
    head = jax.lax.broadcasted_iota(jnp.int32, (8, 256), 0)
    msk = ((lane // 32) == head).astype(jnp.float32)
    m_ref[...] = jnp.dot(alpha, msk, preferred_element_type=jnp.float32) * ve_ref[...]


def _gt2(ex, dg, ve, blk=2048):
    epad = ex.shape[0]
    return pl.pallas_call(
        _gt2_kernel,
        grid=(epad // blk,),
        in_specs=[
            pl.BlockSpec((blk, 256), lambda i: (i, 0)),
            pl.BlockSpec((blk, 256), lambda i: (i, 0)),
            pl.BlockSpec((blk, 256), lambda i: (i, 0)),
        ],
        out_specs=pl.BlockSpec((blk, 256), lambda i: (i, 0)),
        out_shape=jax.ShapeDtypeStruct((epad, 256), jnp.float32),
    )(ex, dg, ve)


def _add_kernel(a_ref, b_ref, o_ref):
    o_ref[...] = a_ref[...] + b_ref[...]


def _add_cols_kernel(a_ref, b_ref, o_ref):
    o_ref[...] = jnp.concatenate(
        [a_ref[:, :76] + b_ref[...], a_ref[:, 76:]], axis=1)


def _add_cols(a, b, npad, blk=4096):
    return pl.pallas_call(
        _add_cols_kernel,
        grid=(pl.cdiv(npad, blk),),
        in_specs=[pl.BlockSpec((blk, 128), lambda i: (i, 0)),
                  pl.BlockSpec((blk, 76), lambda i: (i, 0))],
        out_specs=pl.BlockSpec((blk, 128), lambda i: (i, 0)),
        out_shape=jax.ShapeDtypeStruct((npad, 128), jnp.float32),
    )(a, b)


def _residual_add(a, b, npad=None, blk=4096):
    n, f = a.shape
    npad = n if npad is None else npad
    return pl.pallas_call(
        _add_kernel,
        grid=(pl.cdiv(npad, blk),),
        in_specs=[pl.BlockSpec((blk, f), lambda i: (i, 0)),
                  pl.BlockSpec((blk, f), lambda i: (i, 0))],
        out_specs=pl.BlockSpec((blk, f), lambda i: (i, 0)),
        out_shape=jax.ShapeDtypeStruct((npad, f), jnp.float32),
    )(a, b)


def _head_kernel(xg_ref, agg_ref, cidx_ref, wo_ref, wcf_ref, bcf_ref,
                 w1_ref, b1_ref, w2_ref, b2_ref, wout_ref, bout_ref,
                 out_ref, acc_ref, cnt_ref):
    i = pl.program_id(0)
    nblk = pl.num_programs(0)

    @pl.when(i == 0)
    def _init():
        acc_ref[...] = jnp.zeros_like(acc_ref)
        cnt_ref[...] = jnp.zeros_like(cnt_ref)

    x = jax.nn.softplus(
        xg_ref[...] + jnp.dot(agg_ref[...], wo_ref[...], preferred_element_type=jnp.float32))
    cid = cidx_ref[...]
    oh = (cid == jax.lax.broadcasted_iota(jnp.int32, (1, _N_CRYSTALS), 1)).astype(jnp.float32)
    acc_ref[...] += jax.lax.dot_general(oh, x, (((0,), (0,)), ((), ())),
                                        preferred_element_type=jnp.float32)
    ones = jnp.ones((x.shape[0], 8), jnp.float32)
    cnt_ref[...] += jax.lax.dot_general(oh, ones, (((0,), (0,)), ((), ())),
                                        preferred_element_type=jnp.float32)

    @pl.when(i == nblk - 1)
    def _finish():
        cnt = jnp.clip(cnt_ref[:, 0:1], 1.0, None)
        crys = jax.nn.softplus(acc_ref[...] / cnt)
        crys = jnp.dot(crys, wcf_ref[...], preferred_element_type=jnp.float32) + bcf_ref[...]
        crys = jax.nn.softplus(crys)
        crys = jnp.dot(crys, w1_ref[...], preferred_element_type=jnp.float32) + b1_ref[...]
        crys = jax.nn.softplus(crys)
        crys = jnp.dot(crys, w2_ref[...], preferred_element_type=jnp.float32) + b2_ref[...]
        crys = jax.nn.softplus(crys)
        out_ref[...] = jnp.dot(crys, wout_ref[...], preferred_element_type=jnp.float32) + bout_ref[...]


def _head(xg, agg, cidx, wo, wcf, bcf, fcs, wout, bout, blk=2000):
    n = xg.shape[0]
    return pl.pallas_call(
        _head_kernel,
        grid=(n // blk,),
        in_specs=[
            pl.BlockSpec((blk, 256), lambda i: (i, 0)),
            pl.BlockSpec((blk, 256), lambda i: (i, 0)),
            pl.BlockSpec((blk, 1), lambda i: (i, 0)),
            pl.BlockSpec((256, 256), lambda i: (0, 0)),
            pl.BlockSpec((256, 256), lambda i: (0, 0)),
            pl.BlockSpec((256,), lambda i: (0,)),
            pl.BlockSpec((256, 256), lambda i: (0, 0)),
            pl.BlockSpec((256,), lambda i: (0,)),
            pl.BlockSpec((256, 256), lambda i: (0, 0)),
            pl.BlockSpec((256,), lambda i: (0,)),
            pl.BlockSpec((256, 1), lambda i: (0, 0)),
            pl.BlockSpec((1,), lambda i: (0,)),
        ],
        out_specs=pl.BlockSpec((_N_CRYSTALS, 1), lambda i: (0, 0)),
        out_shape=jax.ShapeDtypeStruct((_N_CRYSTALS, 1), jnp.float32),
        scratch_shapes=[
            pltpu.VMEM((_N_CRYSTALS, 256), jnp.float32),
            pltpu.VMEM((_N_CRYSTALS, 8), jnp.float32),
        ],
    )(xg, agg, cidx.reshape(n, 1).astype(jnp.int32), wo, wcf, bcf,
      fcs[0]['W'], fcs[0]['b'], fcs[1]['W'], fcs[1]['b'], wout, bout)


def _sc_gather(table, idx_pad):
    v, d = table.shape
    bpad = idx_pad.shape[0]
    bw = bpad // _SC_NW
    C = 128
    nch = bw // C
    mesh = plsc.VectorSubcoreMesh(core_axis_name="c", subcore_axis_name="s")

    @functools.partial(
        pl.kernel, mesh=mesh,
        out_type=jax.ShapeDtypeStruct((bpad, d), jnp.float32),
        scratch_types=[
            pltpu.VMEM((bw,), jnp.int32),
            pltpu.VMEM((C, d), jnp.float32),
            pltpu.VMEM((C, d), jnp.float32),
            pltpu.SemaphoreType.DMA,
            pltpu.SemaphoreType.DMA,
        ],
    )
    def k(table_hbm, idx_hbm, out_hbm, idx_all, buf0, buf1, sem0, sem1):
        wid = lax.axis_index("s") * _SC_NC + lax.axis_index("c")
        base = wid * bw
        pltpu.sync_copy(idx_hbm.at[pl.ds(base, bw)], idx_all)
        bufs = ((buf0, sem0), (buf1, sem1))

        def startg(ch, buf, sem):
            pltpu.async_copy(table_hbm.at[idx_all.at[pl.ds(ch * C, C)]], buf, sem)

        startg(0, buf0, sem0)
        startg(1, buf1, sem1)

        def pair(g2, carry):
            for sl, (buf, sem) in enumerate(bufs):
                ch = 2 * g2 + sl
                pltpu.make_async_copy(
                    table_hbm.at[idx_all.at[pl.ds(ch * C, C)]], buf, sem).wait()
                pltpu.sync_copy(buf, out_hbm.at[pl.ds(base + ch * C, C)])

                @pl.when(ch + 2 < nch)
                def _():
                    startg(ch + 2, buf, sem)
            return carry

        lax.fori_loop(0, nch // 2, pair, 0)

    return k(table, idx_pad)


def _sc_scatter_add(msgs, idx_pad, n_rows, nz):
    epad, d = msgs.shape
    dh = d // 2
    rows_alloc = 2048 * nz
    rpp = rows_alloc - 2048
    wr = rpp // _SC_NS
    p_total = -(-n_rows // rpp)
    ew = epad // _SC_NS
    C = 64
    nch = ew // C
    zeros = jnp.zeros((128, dh), jnp.float32)
    mesh = plsc.VectorSubcoreMesh(core_axis_name="c", subcore_axis_name="s")

    @functools.partial(
        pl.kernel, mesh=mesh,
        out_type=jax.ShapeDtypeStruct((p_total * rpp, d), jnp.float32),
        scratch_types=[
            pltpu.VMEM((ew,), jnp.int32),
            pltpu.VMEM((C,), jnp.int32),
            pltpu.VMEM((C,), jnp.int32),
            pltpu.VMEM((C, dh), jnp.float32),
            pltpu.VMEM((C, dh), jnp.float32),
            pltpu.VMEM_SHARED((rows_alloc, dh), jnp.float32),
            pltpu.SemaphoreType.DMA,
            pltpu.SemaphoreType.DMA,
        ],
    )
    def k(msgs_hbm, idx_hbm, z_hbm, out_hbm, idx_all, ix2a, ix2b,
          mbuf0, mbuf1, acc, sem0, sem1):
        core = lax.axis_index("c")
        s = lax.axis_index("s")
        ebase = s * ew
        col0 = core * dh
        pltpu.sync_copy(idx_hbm.at[pl.ds(ebase, ew)], idx_all)
        slots = ((mbuf0, sem0, ix2a), (mbuf1, sem1, ix2b))

        def start_load(ch, mb, sem):
            pltpu.async_copy(
                msgs_hbm.at[pl.ds(ebase + ch * C, C), pl.ds(col0, dh)], mb, sem)

        for p in range(p_total):
            for q in range(nz):
                pltpu.sync_copy(z_hbm, acc.at[pl.ds(s * (128 * nz) + q * 128, 128)])
            plsc.subcore_barrier()
            start_load(0, mbuf0, sem0)
            start_load(1, mbuf1, sem1)

            def chunk_pair(g2, carry):
                for sl, (mb, sem, ix2) in enumerate(slots):
                    ch = 2 * g2 + sl
                    for j in range(C // 16):
                        vj = idx_all[pl.ds(ch * C + j * 16, 16)]
                        local = vj - p * rpp
                        ok = (local >= 0) & (local < rpp)
                        ix2[pl.ds(j * 16, 16)] = jnp.where(ok, local, rpp)
                    pltpu.make_async_copy(
                        msgs_hbm.at[pl.ds(ebase + ch * C, C), pl.ds(col0, dh)],
                        mb, sem).wait()
                    pltpu.sync_copy(mb, acc.at[ix2], add=True)

                    @pl.when(ch + 2 < nch)
                    def _():
                        start_load(ch + 2, mb, sem)
                return carry

            lax.fori_loop(0, nch // 2, chunk_pair, 0)
            plsc.subcore_barrier()
            pltpu.sync_copy(
                acc.at[pl.ds(s * wr, wr)],
                out_hbm.at[pl.ds(p * rpp + s * wr, wr), pl.ds(col0, dh)])
            plsc.subcore_barrier()

    return k(msgs, idx_pad, zeros)


def _pad_idx(ix, bpad, fill):
    ix = ix.astype(jnp.int32)
    return jnp.concatenate([ix, jnp.full((bpad - ix.shape[0],), fill, jnp.int32)])


def kernel(atom_features, pe, spherical, edge_index, line_h, line_edge_index, crystal_atom_idx, params):
    n = pe.shape[0]
    ne = spherical.shape[0]
    nl = line_h.shape[0]
    ep_e = -(-ne // 8192) * 8192
    ep_l = -(-nl // 8192) * 8192
    huge = 1 << 28
    src, dst = edge_index[0], edge_index[1]
    lsrc, ldst = line_edge_index[0], line_edge_index[1]
    dst_g = _pad_idx(dst, ep_e, 0)
    src_g = _pad_idx(src, ep_e, 0)
    dst_s = _pad_idx(dst, ep_e, huge)
    ldst_g = _pad_idx(ldst, ep_l, 0)
    lsrc_g = _pad_idx(lsrc, ep_l, 0)
    ldst_s = _pad_idx(ldst, ep_l, huge)

    nbr = _edge_features(spherical, params['W_edge'], params['b_edge'])
    atom, pe_h = _embed(atom_features, params['embeddings'], params['W_emb'], params['b_emb'],
                        pe, params['W_pe'], params['b_pe'])
    line_h_p = jnp.pad(line_h, (0, ep_l - nl))
    line = _line_features(line_h_p, params['W_line'], params['b_line'])

    for cl, cn in zip(params['line_convs'], params['convs']):
        nd = _sc_gather(nbr, ldst_g)
        ns = _sc_gather(nbr, lsrc_g)
        lmsg = _conv_msgs(nd, ns, line, cl['Wf'], cl['bf'], cl['Ws'], cl['bs'],
                          blk=2048, fx=76, fe=30)
        lagg = jax.ops.segment_sum(lmsg[:nl], ldst.astype(jnp.int32), num_segments=ne)
        nbr = _add_cols(nbr, lagg, npad=ep_e)

        xd = _sc_gather(atom, dst_g)
        xs = _sc_gather(atom, src_g)
        msg = _conv_msgs(xd, xs, nbr, cn['Wf'], cn['bf'], cn['Ws'], cn['bs'],
                         blk=2048, fx=256, fe=76)
        agg = _sc_scatter_add(msg, dst_s, n, nz=6)
        atom = _residual_add(atom, agg)

    xg = _residual_add(atom, pe_h)
    gxd = _sc_gather(xg, dst_g)
    gxs = _sc_gather(xg, src_g)
    ex, ve = _gt1(gxd, gxs, nbr, params['gt']['Wq'], params['gt']['Wk'],
                  params['gt']['Wv'], params['gt']['We'])
    denom = _sc_scatter_add(ex, dst_s, n, nz=6)
    dg = _sc_gather(denom, dst_g)
    msg = _gt2(ex, dg, ve)
    agg = _sc_scatter_add(msg, dst_s, n, nz=6)

    return _head(xg, agg, crystal_atom_idx, params['gt']['Wo'], params['W_cf'],
                 params['b_cf'], params['fcs'], params['W_out'], params['b_out'])


# speedup vs baseline: 3.2101x; 1.0094x over previous
import functools
import jax
import jax.numpy as jnp
from jax import lax
from jax.experimental import pallas as pl
from jax.experimental.pallas import tpu as pltpu
from jax.experimental.pallas import tpu_sc as plsc

_N_CRYSTALS = 128
_SC_NC = 2
_SC_NS = 16
_SC_NW = _SC_NC * _SC_NS


def _lin(n, lo, step):
    return lo + step * jax.lax.broadcasted_iota(jnp.int32, (1, n), 1).astype(jnp.float32)


def _edge_feat_kernel(sph_ref, w_ref, b_ref, out_ref):
    sph = sph_ref[...]
    d0 = sph[:, 0:1]
    d1 = sph[:, 1:2]
    d2 = sph[:, 2:3]
    f0 = jnp.exp(-((d0 - _lin(41, 0.0, 0.2)) ** 2) / (0.2 ** 2))
    f1 = jnp.exp(-((d1 - _lin(17, 0.0, 0.2)) ** 2) / (0.2 ** 2))
    f2 = jnp.exp(-((d2 - _lin(17, -3.2, 0.4)) ** 2) / (0.4 ** 2))
    f3 = (d0 > 8.0).astype(jnp.float32)
    feat = jnp.concatenate([f0, f1, f2, f3], axis=1)
    r = jnp.dot(feat, w_ref[...], preferred_element_type=jnp.float32) + b_ref[...]
    out_ref[...] = jnp.concatenate(
        [r, jnp.zeros((r.shape[0], 52), jnp.float32)], axis=1)


def _edge_features(spherical, w, b):
    ne = spherical.shape[0]
    blk = 2048
    return pl.pallas_call(
        _edge_feat_kernel,
        grid=(pl.cdiv(ne, blk),),
        in_specs=[
            pl.BlockSpec((blk, 3), lambda i: (i, 0)),
            pl.BlockSpec((76, 76), lambda i: (0, 0)),
            pl.BlockSpec((76,), lambda i: (0,)),
        ],
        out_specs=pl.BlockSpec((blk, 128), lambda i: (i, 0)),
        out_shape=jax.ShapeDtypeStruct((ne, 128), jnp.float32),
    )(spherical, w, b)


def _line_feat_kernel(lh_ref, w_ref, b_ref, out_ref):
    d = lh_ref[...][:, None]
    feat = jnp.exp(-((d - _lin(30, -1.4, 0.1)) ** 2) / (0.1 ** 2))
    out_ref[...] = jnp.dot(feat, w_ref[...], preferred_element_type=jnp.float32) + b_ref[...]


def _line_features(line_h, w, b):
    nl = line_h.shape[0]
    blk = 4096
    return pl.pallas_call(
        _line_feat_kernel,
        grid=(nl // blk,),
        in_specs=[
            pl.BlockSpec((blk,), lambda i: (i,)),
            pl.BlockSpec((30, 30), lambda i: (0, 0)),
            pl.BlockSpec((30,), lambda i: (0,)),
        ],
        out_specs=pl.BlockSpec((blk, 30), lambda i: (i, 0)),
        out_shape=jax.ShapeDtypeStruct((nl, 30), jnp.float32),
    )(line_h, w, b)


def _embed_kernel(af_ref, emb_ref, wemb_ref, bemb_ref, pe_ref, wpe_ref, bpe_ref,
                  atom_ref, peh_ref):
    af = af_ref[...]
    oh = (af == jax.lax.broadcasted_iota(jnp.int32, (1, 100), 1)).astype(jnp.float32)
    embw = jnp.dot(emb_ref[...], wemb_ref[...], preferred_element_type=jnp.float32)
    atom_ref[...] = jnp.dot(oh, embw, preferred_element_type=jnp.float32) + bemb_ref[...]
    peh_ref[...] = jnp.dot(pe_ref[...], wpe_ref[...], preferred_element_type=jnp.float32) + bpe_ref[...]


def _embed(atom_features, emb, wemb, bemb, pe, wpe, bpe):
    n = pe.shape[0]
    blk = 2000
    return pl.pallas_call(
        _embed_kernel,
        grid=(n // blk,),
        in_specs=[
            pl.BlockSpec((blk, 1), lambda i: (i, 0)),
            pl.BlockSpec((100, 92), lambda i: (0, 0)),
            pl.BlockSpec((92, 256), lambda i: (0, 0)),
            pl.BlockSpec((256,), lambda i: (0,)),
            pl.BlockSpec((blk, 40), lambda i: (i, 0)),
            pl.BlockSpec((40, 256), lambda i: (0, 0)),
            pl.BlockSpec((256,), lambda i: (0,)),
        ],
        out_specs=[
            pl.BlockSpec((blk, 256), lambda i: (i, 0)),
            pl.BlockSpec((blk, 256), lambda i: (i, 0)),
        ],
        out_shape=[
            jax.ShapeDtypeStruct((n, 256), jnp.float32),
            jax.ShapeDtypeStruct((n, 256), jnp.float32),
        ],
    )(atom_features.astype(jnp.int32), emb, wemb, bemb, pe, wpe, bpe)


def _make_conv_msg_kernel(fx, fe, split38):
    def _conv_msg_kernel(xd_ref, xs_ref, e_ref, wf_ref, bf_ref, ws_ref, bs_ref, *m_refs):
        z = jnp.concatenate([xd_ref[:, :fx], xs_ref[:, :fx], e_ref[:, :fe]], axis=1)
        gate = jax.nn.sigmoid(jnp.dot(z, wf_ref[...], preferred_element_type=jnp.float32) + bf_ref[...])
        core = jax.nn.softplus(jnp.dot(z, ws_ref[...], preferred_element_type=jnp.float32) + bs_ref[...])
        m = gate * core
        if split38:
            m_refs[0][...] = m[:, 0:38]
            m_refs[1][...] = m[:, 38:76]
        else:
            m_refs[0][...] = m
    return _conv_msg_kernel


def _conv_msgs(xd, xs, e, wf, bf, ws, bs, blk, fx, fe, split38=False):
    epad = xd.shape[0]
    fxs = xd.shape[1]
    fes = e.shape[1]
    fz, fo = wf.shape
    if split38:
        out_specs = [pl.BlockSpec((blk, 38), lambda i: (i, 0)),
                     pl.BlockSpec((blk, 38), lambda i: (i, 0))]
        out_shape = [jax.ShapeDtypeStruct((epad, 38), jnp.float32),
                     jax.ShapeDtypeStruct((epad, 38), jnp.float32)]
    else:
        out_specs = pl.BlockSpec((blk, fo), lambda i: (i, 0))
        out_shape = jax.ShapeDtypeStruct((epad, fo), jnp.float32)
    return pl.pallas_call(
        _make_conv_msg_kernel(fx, fe, split38),
        grid=(epad // blk,),
        in_specs=[
            pl.BlockSpec((blk, fxs), lambda i: (i, 0)),
            pl.BlockSpec((blk, fxs), lambda i: (i, 0)),
            pl.BlockSpec((blk, fes), lambda i: (i, 0)),
            pl.BlockSpec((fz, fo), lambda i: (0, 0)),
            pl.BlockSpec((fo,), lambda i: (0,)),
            pl.BlockSpec((fz, fo), lambda i: (0, 0)),
            pl.BlockSpec((fo,), lambda i: (0,)),
        ],
        out_specs=out_specs,
        out_shape=out_shape,
    )(xd, xs, e, wf, bf, ws, bs)


def _gt1_kernel(xd_ref, xs_ref, nbr_ref, wq_ref, wk_ref, wv_ref, we_ref, ex_ref, ve_ref):
    q = jnp.dot(xd_ref[...], wq_ref[...], preferred_element_type=jnp.float32)
    xs = xs_ref[...]
    k = jnp.dot(xs, wk_ref[...], preferred_element_type=jnp.float32)
    v = jnp.dot(xs, wv_ref[...], preferred_element_type=jnp.float32)
    e = jnp.dot(nbr_ref[:, :76], we_ref[...], preferred_element_type=jnp.float32)
    s = q * (k + e)
    lane = jax.lax.broadcasted_iota(jnp.int32, (256, 8), 0)
    head = jax.lax.broadcasted_iota(jnp.int32, (256, 8), 1)
    msk = ((lane // 32) == head).astype(jnp.float32)
    logits = jnp.dot(s, msk, preferred_element_type=jnp.float32) * (1.0 / jnp.sqrt(32.0))
    ex = jnp.exp(logits)
    ex_ref[...] = jnp.concatenate(
        [ex, jnp.zeros((ex.shape[0], 248), jnp.float32)], axis=1)
    mskt = ((jax.lax.broadcasted_iota(jnp.int32, (8, 256), 1) // 32)
            == jax.lax.broadcasted_iota(jnp.int32, (8, 256), 0)).astype(jnp.float32)
    ve_ref[...] = jnp.dot(ex, mskt, preferred_element_type=jnp.float32) * (v + e)


def _gt1(xd, xs, nbr, wq, wk, wv, we, blk=2048):
    epad = xd.shape[0]
    return pl.pallas_call(
        _gt1_kernel,
        grid=(epad // blk,),
        in_specs=[
            pl.BlockSpec((blk, 256), lambda i: (i, 0)),
            pl.BlockSpec((blk, 256), lambda i: (i, 0)),
            pl.BlockSpec((blk, 128), lambda i: (i, 0)),
            pl.BlockSpec((256, 256), lambda i: (0, 0)),
            pl.BlockSpec((256, 256), lambda i: (0, 0)),
            pl.BlockSpec((256, 256), lambda i: (0, 0)),
            pl.BlockSpec((76, 256), lambda i: (0, 0)),
        ],
        out_specs=[
            pl.BlockSpec((blk, 256), lambda i: (i, 0)),
            pl.BlockSpec((blk, 256), lambda i: (i, 0)),
        ],
        out_shape=[
            jax.ShapeDtypeStruct((epad, 256), jnp.float32),
            jax.ShapeDtypeStruct((epad, 256), jnp.float32),
        ],
    )(xd, xs, nbr, wq, wk, wv, we)


def _gt2_kernel(ex_ref, dg_ref, ve_ref, m_ref):
    alpha = ex_ref[:, 0:8] / (dg_ref[:, 0:8] + 1e-9)
    lane = jax.lax.broadcasted_iota(jnp.int32, (8, 256), 1)
    head = jax.lax.broadcasted_iota(jnp.int32, (8, 256), 0)
    msk = ((lane // 32) == head).astype(jnp.float32)
    m_ref[...] = jnp.dot(alpha, msk, preferred_element_type=jnp.float32) * ve_ref[...]


def _gt2(ex, dg, ve, blk=2048):
    epad = ex.shape[0]
    return pl.pallas_call(
        _gt2_kernel,
        grid=(epad // blk,),
        in_specs=[
            pl.BlockSpec((blk, 256), lambda i: (i, 0)),
            pl.BlockSpec((blk, 256), lambda i: (i, 0)),
            pl.BlockSpec((blk, 256), lambda i: (i, 0)),
        ],
        out_specs=pl.BlockSpec((blk, 256), lambda i: (i, 0)),
        out_shape=jax.ShapeDtypeStruct((epad, 256), jnp.float32),
    )(ex, dg, ve)


def _add_kernel(a_ref, b_ref, o_ref):
    o_ref[...] = a_ref[...] + b_ref[...]


def _add_cols_kernel(a_ref, b_ref, o_ref):
    o_ref[...] = jnp.concatenate(
        [a_ref[:, :76] + b_ref[...], a_ref[:, 76:]], axis=1)


def _add_cols(a, b, npad, blk=4096):
    return pl.pallas_call(
        _add_cols_kernel,
        grid=(pl.cdiv(npad, blk),),
        in_specs=[pl.BlockSpec((blk, 128), lambda i: (i, 0)),
                  pl.BlockSpec((blk, 76), lambda i: (i, 0))],
        out_specs=pl.BlockSpec((blk, 128), lambda i: (i, 0)),
        out_shape=jax.ShapeDtypeStruct((npad, 128), jnp.float32),
    )(a, b)


def _residual_add(a, b, npad=None, blk=4096):
    n, f = a.shape
    npad = n if npad is None else npad
    return pl.pallas_call(
        _add_kernel,
        grid=(pl.cdiv(npad, blk),),
        in_specs=[pl.BlockSpec((blk, f), lambda i: (i, 0)),
                  pl.BlockSpec((blk, f), lambda i: (i, 0))],
        out_specs=pl.BlockSpec((blk, f), lambda i: (i, 0)),
        out_shape=jax.ShapeDtypeStruct((npad, f), jnp.float32),
    )(a, b)


def _head_kernel(xg_ref, agg_ref, den_ref, cidx_ref, wo_ref, wcf_ref, bcf_ref,
                 w1_ref, b1_ref, w2_ref, b2_ref, wout_ref, bout_ref,
                 out_ref, acc_ref, cnt_ref):
    i = pl.program_id(0)
    nblk = pl.num_programs(0)

    @pl.when(i == 0)
    def _init():
        acc_ref[...] = jnp.zeros_like(acc_ref)
        cnt_ref[...] = jnp.zeros_like(cnt_ref)

    invd = 1.0 / (den_ref[:, 0:8] + 1e-9)
    mskt = ((jax.lax.broadcasted_iota(jnp.int32, (8, 256), 1) // 32)
            == jax.lax.broadcasted_iota(jnp.int32, (8, 256), 0)).astype(jnp.float32)
    agg = jnp.dot(invd, mskt, preferred_element_type=jnp.float32) * agg_ref[...]
    x = jax.nn.softplus(
        xg_ref[...] + jnp.dot(agg, wo_ref[...], preferred_element_type=jnp.float32))
    cid = cidx_ref[...]
    oh = (cid == jax.lax.broadcasted_iota(jnp.int32, (1, _N_CRYSTALS), 1)).astype(jnp.float32)
    acc_ref[...] += jax.lax.dot_general(oh, x, (((0,), (0,)), ((), ())),
                                        preferred_element_type=jnp.float32)
    ones = jnp.ones((x.shape[0], 8), jnp.float32)
    cnt_ref[...] += jax.lax.dot_general(oh, ones, (((0,), (0,)), ((), ())),
                                        preferred_element_type=jnp.float32)

    @pl.when(i == nblk - 1)
    def _finish():
        cnt = jnp.clip(cnt_ref[:, 0:1], 1.0, None)
        crys = jax.nn.softplus(acc_ref[...] / cnt)
        crys = jnp.dot(crys, wcf_ref[...], preferred_element_type=jnp.float32) + bcf_ref[...]
        crys = jax.nn.softplus(crys)
        crys = jnp.dot(crys, w1_ref[...], preferred_element_type=jnp.float32) + b1_ref[...]
        crys = jax.nn.softplus(crys)
        crys = jnp.dot(crys, w2_ref[...], preferred_element_type=jnp.float32) + b2_ref[...]
        crys = jax.nn.softplus(crys)
        out_ref[...] = jnp.dot(crys, wout_ref[...], preferred_element_type=jnp.float32) + bout_ref[...]


def _head(xg, agg, den, cidx, wo, wcf, bcf, fcs, wout, bout, blk=2000):
    n = xg.shape[0]
    return pl.pallas_call(
        _head_kernel,
        grid=(n // blk,),
        in_specs=[
            pl.BlockSpec((blk, 256), lambda i: (i, 0)),
            pl.BlockSpec((blk, 256), lambda i: (i, 0)),
            pl.BlockSpec((blk, 256), lambda i: (i, 0)),
            pl.BlockSpec((blk, 1), lambda i: (i, 0)),
            pl.BlockSpec((256, 256), lambda i: (0, 0)),
            pl.BlockSpec((256, 256), lambda i: (0, 0)),
            pl.BlockSpec((256,), lambda i: (0,)),
            pl.BlockSpec((256, 256), lambda i: (0, 0)),
            pl.BlockSpec((256,), lambda i: (0,)),
            pl.BlockSpec((256, 256), lambda i: (0, 0)),
            pl.BlockSpec((256,), lambda i: (0,)),
            pl.BlockSpec((256, 1), lambda i: (0, 0)),
            pl.BlockSpec((1,), lambda i: (0,)),
        ],
        out_specs=pl.BlockSpec((_N_CRYSTALS, 1), lambda i: (0, 0)),
        out_shape=jax.ShapeDtypeStruct((_N_CRYSTALS, 1), jnp.float32),
        scratch_shapes=[
            pltpu.VMEM((_N_CRYSTALS, 256), jnp.float32),
            pltpu.VMEM((_N_CRYSTALS, 8), jnp.float32),
        ],
    )(xg, agg, den, cidx.reshape(n, 1).astype(jnp.int32), wo, wcf, bcf,
      fcs[0]['W'], fcs[0]['b'], fcs[1]['W'], fcs[1]['b'], wout, bout)


def _sc_gather(table, idx_pad):
    v, d = table.shape
    bpad = idx_pad.shape[0]
    bw = bpad // _SC_NW
    C = 128
    nch = bw // C
    mesh = plsc.VectorSubcoreMesh(core_axis_name="c", subcore_axis_name="s")

    @functools.partial(
        pl.kernel, mesh=mesh,
        out_type=jax.ShapeDtypeStruct((bpad, d), jnp.float32),
        scratch_types=[
            pltpu.VMEM((bw,), jnp.int32),
            pltpu.VMEM((C, d), jnp.float32),
            pltpu.VMEM((C, d), jnp.float32),
            pltpu.SemaphoreType.DMA,
            pltpu.SemaphoreType.DMA,
        ],
    )
    def k(table_hbm, idx_hbm, out_hbm, idx_all, buf0, buf1, sem0, sem1):
        wid = lax.axis_index("s") * _SC_NC + lax.axis_index("c")
        base = wid * bw
        pltpu.sync_copy(idx_hbm.at[pl.ds(base, bw)], idx_all)
        bufs = ((buf0, sem0), (buf1, sem1))

        def startg(ch, buf, sem):
            pltpu.async_copy(table_hbm.at[idx_all.at[pl.ds(ch * C, C)]], buf, sem)

        startg(0, buf0, sem0)
        startg(1, buf1, sem1)

        def pair(g2, carry):
            for sl, (buf, sem) in enumerate(bufs):
                ch = 2 * g2 + sl
                pltpu.make_async_copy(
                    table_hbm.at[idx_all.at[pl.ds(ch * C, C)]], buf, sem).wait()
                pltpu.sync_copy(buf, out_hbm.at[pl.ds(base + ch * C, C)])

                @pl.when(ch + 2 < nch)
                def _():
                    startg(ch + 2, buf, sem)
            return carry

        lax.fori_loop(0, nch // 2, pair, 0)

    return k(table, idx_pad)


def _sc_scatter_add(msgs, idx_pad, n_rows, nz):
    epad, d = msgs.shape
    dh = d // 2
    rows_alloc = 2048 * nz
    rpp = rows_alloc - 2048
    wr = rpp // _SC_NS
    p_total = -(-n_rows // rpp)
    ew = epad // _SC_NS
    C = 64
    nch = ew // C
    zeros = jnp.zeros((128, dh), jnp.float32)
    mesh = plsc.VectorSubcoreMesh(core_axis_name="c", subcore_axis_name="s")

    @functools.partial(
        pl.kernel, mesh=mesh,
        out_type=jax.ShapeDtypeStruct((p_total * rpp, d), jnp.float32),
        scratch_types=[
            pltpu.VMEM((ew,), jnp.int32),
            pltpu.VMEM((C,), jnp.int32),
            pltpu.VMEM((C,), jnp.int32),
            pltpu.VMEM((C, dh), jnp.float32),
            pltpu.VMEM((C, dh), jnp.float32),
            pltpu.VMEM_SHARED((rows_alloc, dh), jnp.float32),
            pltpu.SemaphoreType.DMA,
            pltpu.SemaphoreType.DMA,
        ],
    )
    def k(msgs_hbm, idx_hbm, z_hbm, out_hbm, idx_all, ix2a, ix2b,
          mbuf0, mbuf1, acc, sem0, sem1):
        core = lax.axis_index("c")
        s = lax.axis_index("s")
        ebase = s * ew
        col0 = core * dh
        pltpu.sync_copy(idx_hbm.at[pl.ds(ebase, ew)], idx_all)
        slots = ((mbuf0, sem0, ix2a), (mbuf1, sem1, ix2b))

        def start_load(ch, mb, sem):
            pltpu.async_copy(
                msgs_hbm.at[pl.ds(ebase + ch * C, C), pl.ds(col0, dh)], mb, sem)

        for p in range(p_total):
            for q in range(nz):
                pltpu.sync_copy(z_hbm, acc.at[pl.ds(s * (128 * nz) + q * 128, 128)])
            plsc.subcore_barrier()
            start_load(0, mbuf0, sem0)
            start_load(1, mbuf1, sem1)

            def chunk_pair(g2, carry):
                for sl, (mb, sem, ix2) in enumerate(slots):
                    ch = 2 * g2 + sl
                    for j in range(C // 16):
                        vj = idx_all[pl.ds(ch * C + j * 16, 16)]
                        local = vj - p * rpp
                        ok = (local >= 0) & (local < rpp)
                        ix2[pl.ds(j * 16, 16)] = jnp.where(ok, local, rpp)
                    pltpu.make_async_copy(
                        msgs_hbm.at[pl.ds(ebase + ch * C, C), pl.ds(col0, dh)],
                        mb, sem).wait()
                    pltpu.sync_copy(mb, acc.at[ix2], add=True)

                    @pl.when(ch + 2 < nch)
                    def _():
                        start_load(ch + 2, mb, sem)
                return carry

            lax.fori_loop(0, nch // 2, chunk_pair, 0)
            plsc.subcore_barrier()
            pltpu.sync_copy(
                acc.at[pl.ds(s * wr, wr)],
                out_hbm.at[pl.ds(p * rpp + s * wr, wr), pl.ds(col0, dh)])
            plsc.subcore_barrier()

    return k(msgs, idx_pad, zeros)


def _sc_scatter_add_narrow(msgs, idx_pad, n_rows, nz):
    epad, d = msgs.shape
    rows_alloc = 2048 * nz
    rpp = rows_alloc - 2048
    wr = rpp // _SC_NS
    p_total = -(-n_rows // rpp)
    ew = epad // _SC_NW
    C = 64
    nch = ew // C
    zeros = jnp.zeros((128, d), jnp.float32)
    mesh = plsc.VectorSubcoreMesh(core_axis_name="c", subcore_axis_name="s")

    @functools.partial(
        pl.kernel, mesh=mesh,
        out_type=jax.ShapeDtypeStruct((_SC_NC, p_total * rpp, d), jnp.float32),
        scratch_types=[
            pltpu.VMEM((ew,), jnp.int32),
            pltpu.VMEM((C,), jnp.int32),
            pltpu.VMEM((C,), jnp.int32),
            pltpu.VMEM((C, d), jnp.float32),
            pltpu.VMEM((C, d), jnp.float32),
            pltpu.VMEM_SHARED((rows_alloc, d), jnp.float32),
            pltpu.SemaphoreType.DMA,
            pltpu.SemaphoreType.DMA,
        ],
    )
    def k(msgs_hbm, idx_hbm, z_hbm, out_hbm, idx_all, ix2a, ix2b,
          mbuf0, mbuf1, acc, sem0, sem1):
        core = lax.axis_index("c")
        s = lax.axis_index("s")
        wid = s * _SC_NC + core
        ebase = wid * ew
        pltpu.sync_copy(idx_hbm.at[pl.ds(ebase, ew)], idx_all)
        slots = ((mbuf0, sem0, ix2a), (mbuf1, sem1, ix2b))

        def start_load(ch, mb, sem):
            pltpu.async_copy(msgs_hbm.at[pl.ds(ebase + ch * C, C)], mb, sem)

        for p in range(p_total):
            for q in range(nz):
                pltpu.sync_copy(z_hbm, acc.at[pl.ds(s * (128 * nz) + q * 128, 128)])
            plsc.subcore_barrier()
            start_load(0, mbuf0, sem0)
            start_load(1, mbuf1, sem1)

            def chunk_pair(g2, carry):
                for sl, (mb, sem, ix2) in enumerate(slots):
                    ch = 2 * g2 + sl
                    for j in range(C // 16):
                        vj = idx_all[pl.ds(ch * C + j * 16, 16)]
                        local = vj - p * rpp
                        ok = (local >= 0) & (local < rpp)
                        ix2[pl.ds(j * 16, 16)] = jnp.where(ok, local, rpp)
                    pltpu.make_async_copy(
                        msgs_hbm.at[pl.ds(ebase + ch * C, C)], mb, sem).wait()
                    pltpu.sync_copy(mb, acc.at[ix2], add=True)

                    @pl.when(ch + 2 < nch)
                    def _():
                        start_load(ch + 2, mb, sem)
                return carry

            lax.fori_loop(0, nch // 2, chunk_pair, 0)
            plsc.subcore_barrier()
            pltpu.sync_copy(
                acc.at[pl.ds(s * wr, wr)],
                out_hbm.at[core, pl.ds(p * rpp + s * wr, wr)])
            plsc.subcore_barrier()

    return k(msgs, idx_pad, zeros)


def _pad_idx(ix, bpad, fill):
    ix = ix.astype(jnp.int32)
    return jnp.concatenate([ix, jnp.full((bpad - ix.shape[0],), fill, jnp.int32)])


def kernel(atom_features, pe, spherical, edge_index, line_h, line_edge_index, crystal_atom_idx, params):
    n = pe.shape[0]
    ne = spherical.shape[0]
    nl = line_h.shape[0]
    ep_e = -(-ne // 8192) * 8192
    ep_l = -(-nl // 8192) * 8192
    huge = 1 << 28
    src, dst = edge_index[0], edge_index[1]
    lsrc, ldst = line_edge_index[0], line_edge_index[1]
    dst_g = _pad_idx(dst, ep_e, 0)
    src_g = _pad_idx(src, ep_e, 0)
    dst_s = _pad_idx(dst, ep_e, huge)
    ldst_g = _pad_idx(ldst, ep_l, 0)
    lsrc_g = _pad_idx(lsrc, ep_l, 0)
    ldst_s = _pad_idx(ldst, ep_l, huge)

    nbr = _edge_features(spherical, params['W_edge'], params['b_edge'])
    atom, pe_h = _embed(atom_features, params['embeddings'], params['W_emb'], params['b_emb'],
                        pe, params['W_pe'], params['b_pe'])
    line_h_p = jnp.pad(line_h, (0, ep_l - nl))
    line = _line_features(line_h_p, params['W_line'], params['b_line'])

    for cl, cn in zip(params['line_convs'], params['convs']):
        nd = _sc_gather(nbr, ldst_g)
        ns = _sc_gather(nbr, lsrc_g)
        lmsg = _conv_msgs(nd, ns, line, cl['Wf'], cl['bf'], cl['Ws'], cl['bs'],
                          blk=2048, fx=76, fe=30)
        lagg = jax.ops.segment_sum(lmsg[:nl], ldst.astype(jnp.int32), num_segments=ne)
        nbr = _add_cols(nbr, lagg, npad=ep_e)

        xd = _sc_gather(atom, dst_g)
        xs = _sc_gather(atom, src_g)
        msg = _conv_msgs(xd, xs, nbr, cn['Wf'], cn['bf'], cn['Ws'], cn['bs'],
                         blk=2048, fx=256, fe=76)
        agg = _sc_scatter_add(msg, dst_s, n, nz=6)
        atom = _residual_add(atom, agg)

    xg = _residual_add(atom, pe_h)
    gxd = _sc_gather(xg, dst_g)
    gxs = _sc_gather(xg, src_g)
    ex, u = _gt1(gxd, gxs, nbr, params['gt']['Wq'], params['gt']['Wk'],
                 params['gt']['Wv'], params['gt']['We'])
    denom = _sc_scatter_add(ex, dst_s, n, nz=6)
    agg = _sc_scatter_add(u, dst_s, n, nz=6)

    return _head(xg, agg, denom, crystal_atom_idx, params['gt']['Wo'], params['W_cf'],
                 params['b_cf'], params['fcs'], params['W_out'], params['b_out'])


# speedup vs baseline: 3.2139x; 1.0012x over previous
import functools
import jax
import jax.numpy as jnp
from jax import lax
from jax.experimental import pallas as pl
from jax.experimental.pallas import tpu as pltpu
from jax.experimental.pallas import tpu_sc as plsc

_N_CRYSTALS = 128
_SC_NC = 2
_SC_NS = 16
_SC_NW = _SC_NC * _SC_NS


def _lin(n, lo, step):
    return lo + step * jax.lax.broadcasted_iota(jnp.int32, (1, n), 1).astype(jnp.float32)


def _edge_feat_kernel(sph_ref, w_ref, b_ref, out_ref):
    sph = sph_ref[...]
    d0 = sph[:, 0:1]
    d1 = sph[:, 1:2]
    d2 = sph[:, 2:3]
    f0 = jnp.exp(-((d0 - _lin(41, 0.0, 0.2)) ** 2) / (0.2 ** 2))
    f1 = jnp.exp(-((d1 - _lin(17, 0.0, 0.2)) ** 2) / (0.2 ** 2))
    f2 = jnp.exp(-((d2 - _lin(17, -3.2, 0.4)) ** 2) / (0.4 ** 2))
    f3 = (d0 > 8.0).astype(jnp.float32)
    feat = jnp.concatenate([f0, f1, f2, f3], axis=1)
    r = jnp.dot(feat, w_ref[...], preferred_element_type=jnp.float32) + b_ref[...]
    out_ref[...] = jnp.concatenate(
        [r, jnp.zeros((r.shape[0], 52), jnp.float32)], axis=1)


def _edge_features(spherical, w, b):
    ne = spherical.shape[0]
    blk = 2048
    return pl.pallas_call(
        _edge_feat_kernel,
        grid=(pl.cdiv(ne, blk),),
        in_specs=[
            pl.BlockSpec((blk, 3), lambda i: (i, 0)),
            pl.BlockSpec((76, 76), lambda i: (0, 0)),
            pl.BlockSpec((76,), lambda i: (0,)),
        ],
        out_specs=pl.BlockSpec((blk, 128), lambda i: (i, 0)),
        out_shape=jax.ShapeDtypeStruct((ne, 128), jnp.float32),
    )(spherical, w, b)


def _line_feat_kernel(lh_ref, w_ref, b_ref, out_ref):
    d = lh_ref[...][:, None]
    feat = jnp.exp(-((d - _lin(30, -1.4, 0.1)) ** 2) / (0.1 ** 2))
    out_ref[...] = jnp.dot(feat, w_ref[...], preferred_element_type=jnp.float32) + b_ref[...]


def _line_features(line_h, w, b):
    nl = line_h.shape[0]
    blk = 4096
    return pl.pallas_call(
        _line_feat_kernel,
        grid=(nl // blk,),
        in_specs=[
            pl.BlockSpec((blk,), lambda i: (i,)),
            pl.BlockSpec((30, 30), lambda i: (0, 0)),
            pl.BlockSpec((30,), lambda i: (0,)),
        ],
        out_specs=pl.BlockSpec((blk, 30), lambda i: (i, 0)),
        out_shape=jax.ShapeDtypeStruct((nl, 30), jnp.float32),
    )(line_h, w, b)


def _embed_kernel(af_ref, emb_ref, wemb_ref, bemb_ref, pe_ref, wpe_ref, bpe_ref,
                  atom_ref, peh_ref):
    af = af_ref[...]
    oh = (af == jax.lax.broadcasted_iota(jnp.int32, (1, 100), 1)).astype(jnp.float32)
    embw = jnp.dot(emb_ref[...], wemb_ref[...], preferred_element_type=jnp.float32)
    atom_ref[...] = jnp.dot(oh, embw, preferred_element_type=jnp.float32) + bemb_ref[...]
    peh_ref[...] = jnp.dot(pe_ref[...], wpe_ref[...], preferred_element_type=jnp.float32) + bpe_ref[...]


def _embed(atom_features, emb, wemb, bemb, pe, wpe, bpe):
    n = pe.shape[0]
    blk = 2000
    return pl.pallas_call(
        _embed_kernel,
        grid=(n // blk,),
        in_specs=[
            pl.BlockSpec((blk, 1), lambda i: (i, 0)),
            pl.BlockSpec((100, 92), lambda i: (0, 0)),
            pl.BlockSpec((92, 256), lambda i: (0, 0)),
            pl.BlockSpec((256,), lambda i: (0,)),
            pl.BlockSpec((blk, 40), lambda i: (i, 0)),
            pl.BlockSpec((40, 256), lambda i: (0, 0)),
            pl.BlockSpec((256,), lambda i: (0,)),
        ],
        out_specs=[
            pl.BlockSpec((blk, 256), lambda i: (i, 0)),
            pl.BlockSpec((blk, 256), lambda i: (i, 0)),
        ],
        out_shape=[
            jax.ShapeDtypeStruct((n, 256), jnp.float32),
            jax.ShapeDtypeStruct((n, 256), jnp.float32),
        ],
    )(atom_features.astype(jnp.int32), emb, wemb, bemb, pe, wpe, bpe)


def _make_conv_msg_kernel(fx, fe):
    def _conv_msg_kernel(xd_ref, xs_ref, e_ref, wf_ref, bf_ref, ws_ref, bs_ref, m_ref):
        z = jnp.concatenate([xd_ref[:, :fx], xs_ref[:, :fx], e_ref[:, :fe]], axis=1)
        gate = jax.nn.sigmoid(jnp.dot(z, wf_ref[...], preferred_element_type=jnp.float32) + bf_ref[...])
        core = jax.nn.softplus(jnp.dot(z, ws_ref[...], preferred_element_type=jnp.float32) + bs_ref[...])
        m_ref[...] = gate * core
    return _conv_msg_kernel


def _conv_msgs(xd, xs, e, wf, bf, ws, bs, blk, fx, fe):
    epad = xd.shape[0]
    fxs = xd.shape[1]
    fes = e.shape[1]
    fz, fo = wf.shape
    out_specs = pl.BlockSpec((blk, fo), lambda i: (i, 0))
    out_shape = jax.ShapeDtypeStruct((epad, fo), jnp.float32)
    return pl.pallas_call(
        _make_conv_msg_kernel(fx, fe),
        grid=(epad // blk,),
        in_specs=[
            pl.BlockSpec((blk, fxs), lambda i: (i, 0)),
            pl.BlockSpec((blk, fxs), lambda i: (i, 0)),
            pl.BlockSpec((blk, fes), lambda i: (i, 0)),
            pl.BlockSpec((fz, fo), lambda i: (0, 0)),
            pl.BlockSpec((fo,), lambda i: (0,)),
            pl.BlockSpec((fz, fo), lambda i: (0, 0)),
            pl.BlockSpec((fo,), lambda i: (0,)),
        ],
        out_specs=out_specs,
        out_shape=out_shape,
    )(xd, xs, e, wf, bf, ws, bs)


def _gt1_kernel(xd_ref, xs_ref, nbr_ref, wq_ref, wk_ref, wv_ref, we_ref, ex_ref, ve_ref):
    q = jnp.dot(xd_ref[...], wq_ref[...], preferred_element_type=jnp.float32)
    xs = xs_ref[...]
    k = jnp.dot(xs, wk_ref[...], preferred_element_type=jnp.float32)
    v = jnp.dot(xs, wv_ref[...], preferred_element_type=jnp.float32)
    e = jnp.dot(nbr_ref[:, :76], we_ref[...], preferred_element_type=jnp.float32)
    s = q * (k + e)
    lane = jax.lax.broadcasted_iota(jnp.int32, (256, 8), 0)
    head = jax.lax.broadcasted_iota(jnp.int32, (256, 8), 1)
    msk = ((lane // 32) == head).astype(jnp.float32)
    logits = jnp.dot(s, msk, preferred_element_type=jnp.float32) * (1.0 / jnp.sqrt(32.0))
    ex = jnp.exp(logits)
    ex_ref[...] = jnp.concatenate(
        [ex, jnp.zeros((ex.shape[0], 248), jnp.float32)], axis=1)
    mskt = ((jax.lax.broadcasted_iota(jnp.int32, (8, 256), 1) // 32)
            == jax.lax.broadcasted_iota(jnp.int32, (8, 256), 0)).astype(jnp.float32)
    ve_ref[...] = jnp.dot(ex, mskt, preferred_element_type=jnp.float32) * (v + e)


def _gt1(xd, xs, nbr, wq, wk, wv, we, blk=2048):
    epad = xd.shape[0]
    return pl.pallas_call(
        _gt1_kernel,
        grid=(epad // blk,),
        in_specs=[
            pl.BlockSpec((blk, 256), lambda i: (i, 0)),
            pl.BlockSpec((blk, 256), lambda i: (i, 0)),
            pl.BlockSpec((blk, 128), lambda i: (i, 0)),
            pl.BlockSpec((256, 256), lambda i: (0, 0)),
            pl.BlockSpec((256, 256), lambda i: (0, 0)),
            pl.BlockSpec((256, 256), lambda i: (0, 0)),
            pl.BlockSpec((76, 256), lambda i: (0, 0)),
        ],
        out_specs=[
            pl.BlockSpec((blk, 256), lambda i: (i, 0)),
            pl.BlockSpec((blk, 256), lambda i: (i, 0)),
        ],
        out_shape=[
            jax.ShapeDtypeStruct((epad, 256), jnp.float32),
            jax.ShapeDtypeStruct((epad, 256), jnp.float32),
        ],
    )(xd, xs, nbr, wq, wk, wv, we)


def _add_kernel(a_ref, b_ref, o_ref):
    o_ref[...] = a_ref[...] + b_ref[...]


def _add_cols_kernel(a_ref, b_ref, o_ref):
    o_ref[...] = jnp.concatenate(
        [a_ref[:, :76] + b_ref[...], a_ref[:, 76:]], axis=1)


def _add_cols(a, b, npad, blk=4096):
    return pl.pallas_call(
        _add_cols_kernel,
        grid=(pl.cdiv(npad, blk),),
        in_specs=[pl.BlockSpec((blk, 128), lambda i: (i, 0)),
                  pl.BlockSpec((blk, 76), lambda i: (i, 0))],
        out_specs=pl.BlockSpec((blk, 128), lambda i: (i, 0)),
        out_shape=jax.ShapeDtypeStruct((npad, 128), jnp.float32),
    )(a, b)


def _residual_add(a, b, npad=None, blk=4096):
    n, f = a.shape
    npad = n if npad is None else npad
    return pl.pallas_call(
        _add_kernel,
        grid=(pl.cdiv(npad, blk),),
        in_specs=[pl.BlockSpec((blk, f), lambda i: (i, 0)),
                  pl.BlockSpec((blk, f), lambda i: (i, 0))],
        out_specs=pl.BlockSpec((blk, f), lambda i: (i, 0)),
        out_shape=jax.ShapeDtypeStruct((npad, f), jnp.float32),
    )(a, b)


def _head_kernel(xg_ref, agg_ref, den_ref, cidx_ref, wo_ref, wcf_ref, bcf_ref,
                 w1_ref, b1_ref, w2_ref, b2_ref, wout_ref, bout_ref,
                 out_ref, acc_ref, cnt_ref):
    i = pl.program_id(0)
    nblk = pl.num_programs(0)

    @pl.when(i == 0)
    def _init():
        acc_ref[...] = jnp.zeros_like(acc_ref)
        cnt_ref[...] = jnp.zeros_like(cnt_ref)

    invd = 1.0 / (den_ref[:, 0:8] + 1e-9)
    mskt = ((jax.lax.broadcasted_iota(jnp.int32, (8, 256), 1) // 32)
            == jax.lax.broadcasted_iota(jnp.int32, (8, 256), 0)).astype(jnp.float32)
    agg = jnp.dot(invd, mskt, preferred_element_type=jnp.float32) * agg_ref[...]
    x = jax.nn.softplus(
        xg_ref[...] + jnp.dot(agg, wo_ref[...], preferred_element_type=jnp.float32))
    cid = cidx_ref[...]
    oh = (cid == jax.lax.broadcasted_iota(jnp.int32, (1, _N_CRYSTALS), 1)).astype(jnp.float32)
    acc_ref[...] += jax.lax.dot_general(oh, x, (((0,), (0,)), ((), ())),
                                        preferred_element_type=jnp.float32)
    ones = jnp.ones((x.shape[0], 8), jnp.float32)
    cnt_ref[...] += jax.lax.dot_general(oh, ones, (((0,), (0,)), ((), ())),
                                        preferred_element_type=jnp.float32)

    @pl.when(i == nblk - 1)
    def _finish():
        cnt = jnp.clip(cnt_ref[:, 0:1], 1.0, None)
        crys = jax.nn.softplus(acc_ref[...] / cnt)
        crys = jnp.dot(crys, wcf_ref[...], preferred_element_type=jnp.float32) + bcf_ref[...]
        crys = jax.nn.softplus(crys)
        crys = jnp.dot(crys, w1_ref[...], preferred_element_type=jnp.float32) + b1_ref[...]
        crys = jax.nn.softplus(crys)
        crys = jnp.dot(crys, w2_ref[...], preferred_element_type=jnp.float32) + b2_ref[...]
        crys = jax.nn.softplus(crys)
        out_ref[...] = jnp.dot(crys, wout_ref[...], preferred_element_type=jnp.float32) + bout_ref[...]


def _head(xg, agg, den, cidx, wo, wcf, bcf, fcs, wout, bout, blk=2000):
    n = xg.shape[0]
    return pl.pallas_call(
        _head_kernel,
        grid=(n // blk,),
        in_specs=[
            pl.BlockSpec((blk, 256), lambda i: (i, 0)),
            pl.BlockSpec((blk, 256), lambda i: (i, 0)),
            pl.BlockSpec((blk, 256), lambda i: (i, 0)),
            pl.BlockSpec((blk, 1), lambda i: (i, 0)),
            pl.BlockSpec((256, 256), lambda i: (0, 0)),
            pl.BlockSpec((256, 256), lambda i: (0, 0)),
            pl.BlockSpec((256,), lambda i: (0,)),
            pl.BlockSpec((256, 256), lambda i: (0, 0)),
            pl.BlockSpec((256,), lambda i: (0,)),
            pl.BlockSpec((256, 256), lambda i: (0, 0)),
            pl.BlockSpec((256,), lambda i: (0,)),
            pl.BlockSpec((256, 1), lambda i: (0, 0)),
            pl.BlockSpec((1,), lambda i: (0,)),
        ],
        out_specs=pl.BlockSpec((_N_CRYSTALS, 1), lambda i: (0, 0)),
        out_shape=jax.ShapeDtypeStruct((_N_CRYSTALS, 1), jnp.float32),
        scratch_shapes=[
            pltpu.VMEM((_N_CRYSTALS, 256), jnp.float32),
            pltpu.VMEM((_N_CRYSTALS, 8), jnp.float32),
        ],
    )(xg, agg, den, cidx.reshape(n, 1).astype(jnp.int32), wo, wcf, bcf,
      fcs[0]['W'], fcs[0]['b'], fcs[1]['W'], fcs[1]['b'], wout, bout)


def _sc_gather(table, idx_pad):
    v, d = table.shape
    bpad = idx_pad.shape[0]
    bw = bpad // _SC_NW
    C = 128
    nch = bw // C
    mesh = plsc.VectorSubcoreMesh(core_axis_name="c", subcore_axis_name="s")

    @functools.partial(
        pl.kernel, mesh=mesh,
        out_type=jax.ShapeDtypeStruct((bpad, d), jnp.float32),
        scratch_types=[
            pltpu.VMEM((bw,), jnp.int32),
            pltpu.VMEM((C, d), jnp.float32),
            pltpu.VMEM((C, d), jnp.float32),
            pltpu.SemaphoreType.DMA,
            pltpu.SemaphoreType.DMA,
        ],
    )
    def k(table_hbm, idx_hbm, out_hbm, idx_all, buf0, buf1, sem0, sem1):
        wid = lax.axis_index("s") * _SC_NC + lax.axis_index("c")
        base = wid * bw
        pltpu.sync_copy(idx_hbm.at[pl.ds(base, bw)], idx_all)
        bufs = ((buf0, sem0), (buf1, sem1))

        def startg(ch, buf, sem):
            pltpu.async_copy(table_hbm.at[idx_all.at[pl.ds(ch * C, C)]], buf, sem)

        startg(0, buf0, sem0)
        startg(1, buf1, sem1)

        def pair(g2, carry):
            for sl, (buf, sem) in enumerate(bufs):
                ch = 2 * g2 + sl
                pltpu.make_async_copy(
                    table_hbm.at[idx_all.at[pl.ds(ch * C, C)]], buf, sem).wait()
                pltpu.sync_copy(buf, out_hbm.at[pl.ds(base + ch * C, C)])

                @pl.when(ch + 2 < nch)
                def _():
                    startg(ch + 2, buf, sem)
            return carry

        lax.fori_loop(0, nch // 2, pair, 0)

    return k(table, idx_pad)


def _sc_scatter_add(msgs, idx_pad, n_rows, nz):
    epad, d = msgs.shape
    dh = d // 2
    rows_alloc = 2048 * nz
    rpp = rows_alloc - 2048
    wr = rpp // _SC_NS
    p_total = -(-n_rows // rpp)
    ew = epad // _SC_NS
    C = 64
    nch = ew // C
    zeros = jnp.zeros((128, dh), jnp.float32)
    mesh = plsc.VectorSubcoreMesh(core_axis_name="c", subcore_axis_name="s")

    @functools.partial(
        pl.kernel, mesh=mesh,
        out_type=jax.ShapeDtypeStruct((p_total * rpp, d), jnp.float32),
        scratch_types=[
            pltpu.VMEM((ew,), jnp.int32),
            pltpu.VMEM((C,), jnp.int32),
            pltpu.VMEM((C,), jnp.int32),
            pltpu.VMEM((C, dh), jnp.float32),
            pltpu.VMEM((C, dh), jnp.float32),
            pltpu.VMEM_SHARED((rows_alloc, dh), jnp.float32),
            pltpu.SemaphoreType.DMA,
            pltpu.SemaphoreType.DMA,
        ],
    )
    def k(msgs_hbm, idx_hbm, z_hbm, out_hbm, idx_all, ix2a, ix2b,
          mbuf0, mbuf1, acc, sem0, sem1):
        core = lax.axis_index("c")
        s = lax.axis_index("s")
        ebase = s * ew
        col0 = core * dh
        pltpu.sync_copy(idx_hbm.at[pl.ds(ebase, ew)], idx_all)
        slots = ((mbuf0, sem0, ix2a), (mbuf1, sem1, ix2b))

        def start_load(ch, mb, sem):
            pltpu.async_copy(
                msgs_hbm.at[pl.ds(ebase + ch * C, C), pl.ds(col0, dh)], mb, sem)

        for p in range(p_total):
            for q in range(nz):
                pltpu.sync_copy(z_hbm, acc.at[pl.ds(s * (128 * nz) + q * 128, 128)])
            plsc.subcore_barrier()
            start_load(0, mbuf0, sem0)
            start_load(1, mbuf1, sem1)

            def chunk_pair(g2, carry):
                for sl, (mb, sem, ix2) in enumerate(slots):
                    ch = 2 * g2 + sl
                    for j in range(C // 16):
                        vj = idx_all[pl.ds(ch * C + j * 16, 16)]
                        local = vj - p * rpp
                        ok = (local >= 0) & (local < rpp)
                        ix2[pl.ds(j * 16, 16)] = jnp.where(ok, local, rpp)
                    pltpu.make_async_copy(
                        msgs_hbm.at[pl.ds(ebase + ch * C, C), pl.ds(col0, dh)],
                        mb, sem).wait()
                    pltpu.sync_copy(mb, acc.at[ix2], add=True)

                    @pl.when(ch + 2 < nch)
                    def _():
                        start_load(ch + 2, mb, sem)
                return carry

            lax.fori_loop(0, nch // 2, chunk_pair, 0)
            plsc.subcore_barrier()
            pltpu.sync_copy(
                acc.at[pl.ds(s * wr, wr)],
                out_hbm.at[pl.ds(p * rpp + s * wr, wr), pl.ds(col0, dh)])
            plsc.subcore_barrier()

    return k(msgs, idx_pad, zeros)


def _pad_idx(ix, bpad, fill):
    ix = ix.astype(jnp.int32)
    return jnp.concatenate([ix, jnp.full((bpad - ix.shape[0],), fill, jnp.int32)])


def kernel(atom_features, pe, spherical, edge_index, line_h, line_edge_index, crystal_atom_idx, params):
    n = pe.shape[0]
    ne = spherical.shape[0]
    nl = line_h.shape[0]
    ep_e = -(-ne // 8192) * 8192
    ep_l = -(-nl // 8192) * 8192
    huge = 1 << 28
    src, dst = edge_index[0], edge_index[1]
    lsrc, ldst = line_edge_index[0], line_edge_index[1]
    dst_g = _pad_idx(dst, ep_e, 0)
    src_g = _pad_idx(src, ep_e, 0)
    dst_s = _pad_idx(dst, ep_e, huge)
    ldst_g = _pad_idx(ldst, ep_l, 0)
    lsrc_g = _pad_idx(lsrc, ep_l, 0)
    ldst_s = _pad_idx(ldst, ep_l, huge)

    nbr = _edge_features(spherical, params['W_edge'], params['b_edge'])
    atom, pe_h = _embed(atom_features, params['embeddings'], params['W_emb'], params['b_emb'],
                        pe, params['W_pe'], params['b_pe'])
    line_h_p = jnp.pad(line_h, (0, ep_l - nl))
    line = _line_features(line_h_p, params['W_line'], params['b_line'])

    for cl, cn in zip(params['line_convs'], params['convs']):
        nd = _sc_gather(nbr, ldst_g)
        ns = _sc_gather(nbr, lsrc_g)
        lmsg = _conv_msgs(nd, ns, line, cl['Wf'], cl['bf'], cl['Ws'], cl['bs'],
                          blk=2048, fx=76, fe=30)
        lagg = jax.ops.segment_sum(lmsg[:nl], ldst.astype(jnp.int32), num_segments=ne)
        nbr = _add_cols(nbr, lagg, npad=ep_e)

        xd = _sc_gather(atom, dst_g)
        xs = _sc_gather(atom, src_g)
        msg = _conv_msgs(xd, xs, nbr, cn['Wf'], cn['bf'], cn['Ws'], cn['bs'],
                         blk=2048, fx=256, fe=76)
        agg = _sc_scatter_add(msg, dst_s, n, nz=6)
        atom = _residual_add(atom, agg)

    xg = _residual_add(atom, pe_h)
    gxd = _sc_gather(xg, dst_g)
    gxs = _sc_gather(xg, src_g)
    ex, u = _gt1(gxd, gxs, nbr, params['gt']['Wq'], params['gt']['Wk'],
                 params['gt']['Wv'], params['gt']['We'])
    denom = _sc_scatter_add(ex, dst_s, n, nz=6)
    agg = _sc_scatter_add(u, dst_s, n, nz=6)

    return _head(xg, agg, denom, crystal_atom_idx, params['gt']['Wo'], params['W_cf'],
                 params['b_cf'], params['fcs'], params['W_out'], params['b_out'])


# speedup vs baseline: 3.2193x; 1.0017x over previous
import functools
import jax
import jax.numpy as jnp
from jax import lax
from jax.experimental import pallas as pl
from jax.experimental.pallas import tpu as pltpu
from jax.experimental.pallas import tpu_sc as plsc

_N_CRYSTALS = 128
_SC_NC = 2
_SC_NS = 16
_SC_NW = _SC_NC * _SC_NS


def _lin(n, lo, step):
    return lo + step * jax.lax.broadcasted_iota(jnp.int32, (1, n), 1).astype(jnp.float32)


def _edge_feat_kernel(sph_ref, w_ref, b_ref, out_ref):
    sph = sph_ref[...]
    d0 = sph[:, 0:1]
    d1 = sph[:, 1:2]
    d2 = sph[:, 2:3]
    f0 = jnp.exp(-((d0 - _lin(41, 0.0, 0.2)) ** 2) / (0.2 ** 2))
    f1 = jnp.exp(-((d1 - _lin(17, 0.0, 0.2)) ** 2) / (0.2 ** 2))
    f2 = jnp.exp(-((d2 - _lin(17, -3.2, 0.4)) ** 2) / (0.4 ** 2))
    f3 = (d0 > 8.0).astype(jnp.float32)
    feat = jnp.concatenate([f0, f1, f2, f3], axis=1)
    r = jnp.dot(feat, w_ref[...], preferred_element_type=jnp.float32) + b_ref[...]
    out_ref[...] = jnp.concatenate(
        [r, jnp.zeros((r.shape[0], 52), jnp.float32)], axis=1)


def _edge_features(spherical, w, b):
    ne = spherical.shape[0]
    blk = 2048
    return pl.pallas_call(
        _edge_feat_kernel,
        grid=(pl.cdiv(ne, blk),),
        in_specs=[
            pl.BlockSpec((blk, 3), lambda i: (i, 0)),
            pl.BlockSpec((76, 76), lambda i: (0, 0)),
            pl.BlockSpec((76,), lambda i: (0,)),
        ],
        out_specs=pl.BlockSpec((blk, 128), lambda i: (i, 0)),
        out_shape=jax.ShapeDtypeStruct((ne, 128), jnp.float32),
    )(spherical, w, b)


def _line_feat_kernel(lh_ref, w_ref, b_ref, out_ref):
    d = lh_ref[...][:, None]
    feat = jnp.exp(-((d - _lin(30, -1.4, 0.1)) ** 2) / (0.1 ** 2))
    out_ref[...] = jnp.dot(feat, w_ref[...], preferred_element_type=jnp.float32) + b_ref[...]


def _line_features(line_h, w, b):
    nl = line_h.shape[0]
    blk = 4096
    return pl.pallas_call(
        _line_feat_kernel,
        grid=(nl // blk,),
        in_specs=[
            pl.BlockSpec((blk,), lambda i: (i,)),
            pl.BlockSpec((30, 30), lambda i: (0, 0)),
            pl.BlockSpec((30,), lambda i: (0,)),
        ],
        out_specs=pl.BlockSpec((blk, 30), lambda i: (i, 0)),
        out_shape=jax.ShapeDtypeStruct((nl, 30), jnp.float32),
    )(line_h, w, b)


def _embed_kernel(af_ref, emb_ref, wemb_ref, bemb_ref, pe_ref, wpe_ref, bpe_ref,
                  atom_ref, peh_ref):
    af = af_ref[...]
    oh = (af == jax.lax.broadcasted_iota(jnp.int32, (1, 100), 1)).astype(jnp.float32)
    embw = jnp.dot(emb_ref[...], wemb_ref[...], preferred_element_type=jnp.float32)
    atom_ref[...] = jnp.dot(oh, embw, preferred_element_type=jnp.float32) + bemb_ref[...]
    peh_ref[...] = jnp.dot(pe_ref[...], wpe_ref[...], preferred_element_type=jnp.float32) + bpe_ref[...]


def _embed(atom_features, emb, wemb, bemb, pe, wpe, bpe):
    n = pe.shape[0]
    blk = 2000
    return pl.pallas_call(
        _embed_kernel,
        grid=(n // blk,),
        in_specs=[
            pl.BlockSpec((blk, 1), lambda i: (i, 0)),
            pl.BlockSpec((100, 92), lambda i: (0, 0)),
            pl.BlockSpec((92, 256), lambda i: (0, 0)),
            pl.BlockSpec((256,), lambda i: (0,)),
            pl.BlockSpec((blk, 40), lambda i: (i, 0)),
            pl.BlockSpec((40, 256), lambda i: (0, 0)),
            pl.BlockSpec((256,), lambda i: (0,)),
        ],
        out_specs=[
            pl.BlockSpec((blk, 256), lambda i: (i, 0)),
            pl.BlockSpec((blk, 256), lambda i: (i, 0)),
        ],
        out_shape=[
            jax.ShapeDtypeStruct((n, 256), jnp.float32),
            jax.ShapeDtypeStruct((n, 256), jnp.float32),
        ],
    )(atom_features.astype(jnp.int32), emb, wemb, bemb, pe, wpe, bpe)


def _make_conv_msg_kernel(fx, fe):
    def _conv_msg_kernel(xd_ref, xs_ref, e_ref, wf_ref, bf_ref, ws_ref, bs_ref, m_ref):
        z = jnp.concatenate([xd_ref[:, :fx], xs_ref[:, :fx], e_ref[:, :fe]], axis=1)
        gate = jax.nn.sigmoid(jnp.dot(z, wf_ref[...], preferred_element_type=jnp.float32) + bf_ref[...])
        core = jax.nn.softplus(jnp.dot(z, ws_ref[...], preferred_element_type=jnp.float32) + bs_ref[...])
        m_ref[...] = gate * core
    return _conv_msg_kernel


def _conv_msgs(xd, xs, e, wf, bf, ws, bs, blk, fx, fe):
    epad = xd.shape[0]
    fxs = xd.shape[1]
    fes = e.shape[1]
    fz, fo = wf.shape
    out_specs = pl.BlockSpec((blk, fo), lambda i: (i, 0))
    out_shape = jax.ShapeDtypeStruct((epad, fo), jnp.float32)
    return pl.pallas_call(
        _make_conv_msg_kernel(fx, fe),
        grid=(epad // blk,),
        in_specs=[
            pl.BlockSpec((blk, fxs), lambda i: (i, 0)),
            pl.BlockSpec((blk, fxs), lambda i: (i, 0)),
            pl.BlockSpec((blk, fes), lambda i: (i, 0)),
            pl.BlockSpec((fz, fo), lambda i: (0, 0)),
            pl.BlockSpec((fo,), lambda i: (0,)),
            pl.BlockSpec((fz, fo), lambda i: (0, 0)),
            pl.BlockSpec((fo,), lambda i: (0,)),
        ],
        out_specs=out_specs,
        out_shape=out_shape,
    )(xd, xs, e, wf, bf, ws, bs)


def _gt1_kernel(xd_ref, xs_ref, nbr_ref, wq_ref, wk_ref, wv_ref, we_ref, ex_ref, ve_ref):
    q = jnp.dot(xd_ref[...], wq_ref[...], preferred_element_type=jnp.float32)
    xs = xs_ref[...]
    k = jnp.dot(xs, wk_ref[...], preferred_element_type=jnp.float32)
    v = jnp.dot(xs, wv_ref[...], preferred_element_type=jnp.float32)
    e = jnp.dot(nbr_ref[:, :76], we_ref[...], preferred_element_type=jnp.float32)
    s = q * (k + e)
    lane = jax.lax.broadcasted_iota(jnp.int32, (256, 8), 0)
    head = jax.lax.broadcasted_iota(jnp.int32, (256, 8), 1)
    msk = ((lane // 32) == head).astype(jnp.float32)
    logits = jnp.dot(s, msk, preferred_element_type=jnp.float32) * (1.0 / jnp.sqrt(32.0))
    ex = jnp.exp(logits)
    ex_ref[...] = jnp.concatenate(
        [ex, jnp.zeros((ex.shape[0], 248), jnp.float32)], axis=1)
    mskt = ((jax.lax.broadcasted_iota(jnp.int32, (8, 256), 1) // 32)
            == jax.lax.broadcasted_iota(jnp.int32, (8, 256), 0)).astype(jnp.float32)
    ve_ref[...] = jnp.dot(ex, mskt, preferred_element_type=jnp.float32) * (v + e)


def _gt1(xd, xs, nbr, wq, wk, wv, we, blk=2048):
    epad = xd.shape[0]
    return pl.pallas_call(
        _gt1_kernel,
        grid=(epad // blk,),
        in_specs=[
            pl.BlockSpec((blk, 256), lambda i: (i, 0)),
            pl.BlockSpec((blk, 256), lambda i: (i, 0)),
            pl.BlockSpec((blk, 128), lambda i: (i, 0)),
            pl.BlockSpec((256, 256), lambda i: (0, 0)),
            pl.BlockSpec((256, 256), lambda i: (0, 0)),
            pl.BlockSpec((256, 256), lambda i: (0, 0)),
            pl.BlockSpec((76, 256), lambda i: (0, 0)),
        ],
        out_specs=[
            pl.BlockSpec((blk, 256), lambda i: (i, 0)),
            pl.BlockSpec((blk, 256), lambda i: (i, 0)),
        ],
        out_shape=[
            jax.ShapeDtypeStruct((epad, 256), jnp.float32),
            jax.ShapeDtypeStruct((epad, 256), jnp.float32),
        ],
    )(xd, xs, nbr, wq, wk, wv, we)


def _add_kernel(a_ref, b_ref, o_ref):
    o_ref[...] = a_ref[...] + b_ref[...]


def _add_cols_kernel(a_ref, b_ref, o_ref):
    o_ref[...] = jnp.concatenate(
        [a_ref[:, :76] + b_ref[...], a_ref[:, 76:]], axis=1)


def _add_cols(a, b, npad, blk=4096):
    return pl.pallas_call(
        _add_cols_kernel,
        grid=(pl.cdiv(npad, blk),),
        in_specs=[pl.BlockSpec((blk, 128), lambda i: (i, 0)),
                  pl.BlockSpec((blk, 76), lambda i: (i, 0))],
        out_specs=pl.BlockSpec((blk, 128), lambda i: (i, 0)),
        out_shape=jax.ShapeDtypeStruct((npad, 128), jnp.float32),
    )(a, b)


def _residual_add(a, b, npad=None, blk=4096):
    n, f = a.shape
    npad = n if npad is None else npad
    return pl.pallas_call(
        _add_kernel,
        grid=(pl.cdiv(npad, blk),),
        in_specs=[pl.BlockSpec((blk, f), lambda i: (i, 0)),
                  pl.BlockSpec((blk, f), lambda i: (i, 0))],
        out_specs=pl.BlockSpec((blk, f), lambda i: (i, 0)),
        out_shape=jax.ShapeDtypeStruct((npad, f), jnp.float32),
    )(a, b)


def _head_kernel(xg_ref, agg_ref, den_ref, cidx_ref, wo_ref, wcf_ref, bcf_ref,
                 w1_ref, b1_ref, w2_ref, b2_ref, wout_ref, bout_ref,
                 out_ref, acc_ref, cnt_ref):
    i = pl.program_id(0)
    nblk = pl.num_programs(0)

    @pl.when(i == 0)
    def _init():
        acc_ref[...] = jnp.zeros_like(acc_ref)
        cnt_ref[...] = jnp.zeros_like(cnt_ref)

    invd = 1.0 / (den_ref[:, 0:8] + 1e-9)
    mskt = ((jax.lax.broadcasted_iota(jnp.int32, (8, 256), 1) // 32)
            == jax.lax.broadcasted_iota(jnp.int32, (8, 256), 0)).astype(jnp.float32)
    agg = jnp.dot(invd, mskt, preferred_element_type=jnp.float32) * agg_ref[...]
    x = jax.nn.softplus(
        xg_ref[...] + jnp.dot(agg, wo_ref[...], preferred_element_type=jnp.float32))
    cid = cidx_ref[...]
    oh = (cid == jax.lax.broadcasted_iota(jnp.int32, (1, _N_CRYSTALS), 1)).astype(jnp.float32)
    acc_ref[...] += jax.lax.dot_general(oh, x, (((0,), (0,)), ((), ())),
                                        preferred_element_type=jnp.float32)
    ones = jnp.ones((x.shape[0], 8), jnp.float32)
    cnt_ref[...] += jax.lax.dot_general(oh, ones, (((0,), (0,)), ((), ())),
                                        preferred_element_type=jnp.float32)

    @pl.when(i == nblk - 1)
    def _finish():
        cnt = jnp.clip(cnt_ref[:, 0:1], 1.0, None)
        crys = jax.nn.softplus(acc_ref[...] / cnt)
        crys = jnp.dot(crys, wcf_ref[...], preferred_element_type=jnp.float32) + bcf_ref[...]
        crys = jax.nn.softplus(crys)
        crys = jnp.dot(crys, w1_ref[...], preferred_element_type=jnp.float32) + b1_ref[...]
        crys = jax.nn.softplus(crys)
        crys = jnp.dot(crys, w2_ref[...], preferred_element_type=jnp.float32) + b2_ref[...]
        crys = jax.nn.softplus(crys)
        out_ref[...] = jnp.dot(crys, wout_ref[...], preferred_element_type=jnp.float32) + bout_ref[...]


def _head(xg, agg, den, cidx, wo, wcf, bcf, fcs, wout, bout, blk=2000):
    n = xg.shape[0]
    return pl.pallas_call(
        _head_kernel,
        grid=(n // blk,),
        in_specs=[
            pl.BlockSpec((blk, 256), lambda i: (i, 0)),
            pl.BlockSpec((blk, 256), lambda i: (i, 0)),
            pl.BlockSpec((blk, 256), lambda i: (i, 0)),
            pl.BlockSpec((blk, 1), lambda i: (i, 0)),
            pl.BlockSpec((256, 256), lambda i: (0, 0)),
            pl.BlockSpec((256, 256), lambda i: (0, 0)),
            pl.BlockSpec((256,), lambda i: (0,)),
            pl.BlockSpec((256, 256), lambda i: (0, 0)),
            pl.BlockSpec((256,), lambda i: (0,)),
            pl.BlockSpec((256, 256), lambda i: (0, 0)),
            pl.BlockSpec((256,), lambda i: (0,)),
            pl.BlockSpec((256, 1), lambda i: (0, 0)),
            pl.BlockSpec((1,), lambda i: (0,)),
        ],
        out_specs=pl.BlockSpec((_N_CRYSTALS, 1), lambda i: (0, 0)),
        out_shape=jax.ShapeDtypeStruct((_N_CRYSTALS, 1), jnp.float32),
        scratch_shapes=[
            pltpu.VMEM((_N_CRYSTALS, 256), jnp.float32),
            pltpu.VMEM((_N_CRYSTALS, 8), jnp.float32),
        ],
    )(xg, agg, den, cidx.reshape(n, 1).astype(jnp.int32), wo, wcf, bcf,
      fcs[0]['W'], fcs[0]['b'], fcs[1]['W'], fcs[1]['b'], wout, bout)


def _sc_gather(table, idx_pad):
    v, d = table.shape
    bpad = idx_pad.shape[0]
    bw = bpad // _SC_NW
    C = 64 if d > 128 else 128
    nch = bw // C
    mesh = plsc.VectorSubcoreMesh(core_axis_name="c", subcore_axis_name="s")

    @functools.partial(
        pl.kernel, mesh=mesh,
        out_type=jax.ShapeDtypeStruct((bpad, d), jnp.float32),
        scratch_types=[
            pltpu.VMEM((bw,), jnp.int32),
            pltpu.VMEM((C, d), jnp.float32),
            pltpu.VMEM((C, d), jnp.float32),
            pltpu.VMEM((C, d), jnp.float32),
            pltpu.VMEM((C, d), jnp.float32),
            pltpu.SemaphoreType.DMA,
            pltpu.SemaphoreType.DMA,
            pltpu.SemaphoreType.DMA,
            pltpu.SemaphoreType.DMA,
            pltpu.SemaphoreType.DMA,
            pltpu.SemaphoreType.DMA,
            pltpu.SemaphoreType.DMA,
            pltpu.SemaphoreType.DMA,
        ],
    )
    def k(table_hbm, idx_hbm, out_hbm, idx_all, b0, b1, b2, b3,
          gs0, gs1, gs2, gs3, os0, os1, os2, os3):
        wid = lax.axis_index("s") * _SC_NC + lax.axis_index("c")
        base = wid * bw
        pltpu.sync_copy(idx_hbm.at[pl.ds(base, bw)], idx_all)
        bufs = (b0, b1, b2, b3)
        gsems = (gs0, gs1, gs2, gs3)
        osems = (os0, os1, os2, os3)

        def startg(ch, sl):
            pltpu.async_copy(table_hbm.at[idx_all.at[pl.ds(ch * C, C)]],
                             bufs[sl], gsems[sl])

        def waitg(ch, sl):
            pltpu.make_async_copy(table_hbm.at[idx_all.at[pl.ds(ch * C, C)]],
                                  bufs[sl], gsems[sl]).wait()

        def starto(ch, sl):
            pltpu.async_copy(bufs[sl], out_hbm.at[pl.ds(base + ch * C, C)],
                             osems[sl])

        def waito(ch, sl):
            pltpu.make_async_copy(bufs[sl], out_hbm.at[pl.ds(base + ch * C, C)],
                                  osems[sl]).wait()

        startg(0, 0)
        startg(1, 1)

        def body(g, carry):
            for s in range(4):
                ch = 4 * g + s
                sl_next = (s + 2) % 4
                @pl.when(ch + 2 < nch)
                def _():
                    @pl.when(ch >= 2)
                    def _():
                        waito(ch - 2, sl_next)
                    startg(ch + 2, sl_next)
                waitg(ch, s)
                starto(ch, s)
            return carry

        lax.fori_loop(0, nch // 4, body, 0)
        for t in (4, 3, 2, 1):
            waito(nch - t, (nch - t) % 4)

    return k(table, idx_pad)


def _sc_scatter_add(msgs, idx_pad, n_rows, nz):
    epad, d = msgs.shape
    dh = d // 2
    rows_alloc = 2048 * nz
    rpp = rows_alloc - 2048
    wr = rpp // _SC_NS
    p_total = -(-n_rows // rpp)
    ew = epad // _SC_NS
    C = 64
    nch = ew // C
    zeros = jnp.zeros((128, dh), jnp.float32)
    mesh = plsc.VectorSubcoreMesh(core_axis_name="c", subcore_axis_name="s")

    @functools.partial(
        pl.kernel, mesh=mesh,
        out_type=jax.ShapeDtypeStruct((p_total * rpp, d), jnp.float32),
        scratch_types=[
            pltpu.VMEM((ew,), jnp.int32),
            pltpu.VMEM((C,), jnp.int32),
            pltpu.VMEM((C,), jnp.int32),
            pltpu.VMEM((C, dh), jnp.float32),
            pltpu.VMEM((C, dh), jnp.float32),
            pltpu.VMEM_SHARED((rows_alloc, dh), jnp.float32),
            pltpu.SemaphoreType.DMA,
            pltpu.SemaphoreType.DMA,
        ],
    )
    def k(msgs_hbm, idx_hbm, z_hbm, out_hbm, idx_all, ix2a, ix2b,
          mbuf0, mbuf1, acc, sem0, sem1):
        core = lax.axis_index("c")
        s = lax.axis_index("s")
        ebase = s * ew
        col0 = core * dh
        pltpu.sync_copy(idx_hbm.at[pl.ds(ebase, ew)], idx_all)
        slots = ((mbuf0, sem0, ix2a), (mbuf1, sem1, ix2b))

        def start_load(ch, mb, sem):
            pltpu.async_copy(
                msgs_hbm.at[pl.ds(ebase + ch * C, C), pl.ds(col0, dh)], mb, sem)

        for p in range(p_total):
            for q in range(nz):
                pltpu.sync_copy(z_hbm, acc.at[pl.ds(s * (128 * nz) + q * 128, 128)])
            plsc.subcore_barrier()
            start_load(0, mbuf0, sem0)
            start_load(1, mbuf1, sem1)

            def chunk_pair(g2, carry):
                for sl, (mb, sem, ix2) in enumerate(slots):
                    ch = 2 * g2 + sl
                    for j in range(C // 16):
                        vj = idx_all[pl.ds(ch * C + j * 16, 16)]
                        local = vj - p * rpp
                        ok = (local >= 0) & (local < rpp)
                        ix2[pl.ds(j * 16, 16)] = jnp.where(ok, local, rpp)
                    pltpu.make_async_copy(
                        msgs_hbm.at[pl.ds(ebase + ch * C, C), pl.ds(col0, dh)],
                        mb, sem).wait()
                    pltpu.sync_copy(mb, acc.at[ix2], add=True)

                    @pl.when(ch + 2 < nch)
                    def _():
                        start_load(ch + 2, mb, sem)
                return carry

            lax.fori_loop(0, nch // 2, chunk_pair, 0)
            plsc.subcore_barrier()
            pltpu.sync_copy(
                acc.at[pl.ds(s * wr, wr)],
                out_hbm.at[pl.ds(p * rpp + s * wr, wr), pl.ds(col0, dh)])
            plsc.subcore_barrier()

    return k(msgs, idx_pad, zeros)


def _pad_idx(ix, bpad, fill):
    ix = ix.astype(jnp.int32)
    return jnp.concatenate([ix, jnp.full((bpad - ix.shape[0],), fill, jnp.int32)])


def kernel(atom_features, pe, spherical, edge_index, line_h, line_edge_index, crystal_atom_idx, params):
    n = pe.shape[0]
    ne = spherical.shape[0]
    nl = line_h.shape[0]
    ep_e = -(-ne // 8192) * 8192
    ep_l = -(-nl // 8192) * 8192
    huge = 1 << 28
    src, dst = edge_index[0], edge_index[1]
    lsrc, ldst = line_edge_index[0], line_edge_index[1]
    dst_g = _pad_idx(dst, ep_e, 0)
    src_g = _pad_idx(src, ep_e, 0)
    dst_s = _pad_idx(dst, ep_e, huge)
    ldst_g = _pad_idx(ldst, ep_l, 0)
    lsrc_g = _pad_idx(lsrc, ep_l, 0)
    ldst_s = _pad_idx(ldst, ep_l, huge)

    nbr = _edge_features(spherical, params['W_edge'], params['b_edge'])
    atom, pe_h = _embed(atom_features, params['embeddings'], params['W_emb'], params['b_emb'],
                        pe, params['W_pe'], params['b_pe'])
    line_h_p = jnp.pad(line_h, (0, ep_l - nl))
    line = _line_features(line_h_p, params['W_line'], params['b_line'])

    for cl, cn in zip(params['line_convs'], params['convs']):
        nd = _sc_gather(nbr, ldst_g)
        ns = _sc_gather(nbr, lsrc_g)
        lmsg = _conv_msgs(nd, ns, line, cl['Wf'], cl['bf'], cl['Ws'], cl['bs'],
                          blk=2048, fx=76, fe=30)
        lagg = jax.ops.segment_sum(lmsg[:nl], ldst.astype(jnp.int32), num_segments=ne)
        nbr = _add_cols(nbr, lagg, npad=ep_e)

        xd = _sc_gather(atom, dst_g)
        xs = _sc_gather(atom, src_g)
        msg = _conv_msgs(xd, xs, nbr, cn['Wf'], cn['bf'], cn['Ws'], cn['bs'],
                         blk=2048, fx=256, fe=76)
        agg = _sc_scatter_add(msg, dst_s, n, nz=6)
        atom = _residual_add(atom, agg)

    xg = _residual_add(atom, pe_h)
    gxd = _sc_gather(xg, dst_g)
    gxs = _sc_gather(xg, src_g)
    ex, u = _gt1(gxd, gxs, nbr, params['gt']['Wq'], params['gt']['Wk'],
                 params['gt']['Wv'], params['gt']['We'])
    denom = _sc_scatter_add(ex, dst_s, n, nz=6)
    agg = _sc_scatter_add(u, dst_s, n, nz=6)

    return _head(xg, agg, denom, crystal_atom_idx, params['gt']['Wo'], params['W_cf'],
                 params['b_cf'], params['fcs'], params['W_out'], params['b_out'])
